# Initial kernel scaffold; baseline (speedup 1.0000x reference)
import jax
import jax.numpy as jnp
from jax.experimental import pallas as pl


def kernel(node_features, edge_features, senders, receivers, params):
    raise NotImplementedError("write your pallas kernel here")


# speedup vs baseline: 2.1655x; 2.1655x over previous
import functools

import jax
import jax.numpy as jnp
from jax import lax
from jax.experimental import pallas as pl
from jax.experimental.pallas import tpu as pltpu
from jax.experimental.pallas import tpu_sc as plsc

N = 10000
E = 320000
D = 128

NC = 2
NS = 16
NW = NC * NS

C = 128
NB = E // C
ITERS = -(-NB // NW)

PREC = jax.lax.Precision.HIGHEST


def _ln(y, g, beta):
    mu = jnp.mean(y, axis=-1, keepdims=True)
    var = jnp.mean((y - mu) * (y - mu), axis=-1, keepdims=True)
    return (y - mu) * lax.rsqrt(var + 1e-5) * g + beta


def _dot(a, b):
    return jnp.dot(a, b, preferred_element_type=jnp.float32, precision=PREC)


def _row_spec(rows, cols):
    return pl.BlockSpec((rows, cols), lambda i: (i, 0))


def _full_spec(shape):
    nd = len(shape)
    return pl.BlockSpec(shape, lambda i: (0,) * nd)


def _weight_args(p, din):
    args = [p['w1'], p['b1'].reshape(1, -1), p['w2'], p['b2'].reshape(1, -1)]
    if 'g' in p:
        args += [p['g'].reshape(1, -1), p['beta'].reshape(1, -1)]
    specs = [_full_spec(a.shape) for a in args]
    return args, specs


def _mlp_block(x, w1, b1, w2, b2, g=None, beta=None, extra=None):
    x1 = _dot(x, w1) + b1
    if extra is not None:
        x1 = x1 + extra
    h = jnp.maximum(x1, 0.0)
    y = _dot(h, w2) + b2
    if g is not None:
        y = _ln(y, g, beta)
    return y


def _node_encode(x, enc, w1s, w1r):
    rows = 1000

    def body(x_ref, w1_ref, b1_ref, w2_ref, b2_ref, g_ref, beta_ref,
             ws_ref, wr_ref, v_ref, p_ref, q_ref):
        v = _mlp_block(x_ref[...], w1_ref[...], b1_ref[...], w2_ref[...],
                       b2_ref[...], g_ref[...], beta_ref[...])
        v_ref[...] = v
        p_ref[...] = _dot(v, ws_ref[...])
        q_ref[...] = _dot(v, wr_ref[...])

    wargs, wspecs = _weight_args(enc, x.shape[1])
    out_sds = jax.ShapeDtypeStruct((N, D), jnp.float32)
    return pl.pallas_call(
        body,
        grid=(N // rows,),
        in_specs=[_row_spec(rows, x.shape[1])] + wspecs
        + [_full_spec((D, D)), _full_spec((D, D))],
        out_specs=[_row_spec(rows, D)] * 3,
        out_shape=[out_sds] * 3,
    )(x, *wargs, w1s, w1r)


def _edge_encode(x, enc):
    rows = 2000

    def body(x_ref, w1_ref, b1_ref, w2_ref, b2_ref, g_ref, beta_ref, o_ref):
        o_ref[...] = _mlp_block(x_ref[...], w1_ref[...], b1_ref[...],
                                w2_ref[...], b2_ref[...], g_ref[...],
                                beta_ref[...])

    wargs, wspecs = _weight_args(enc, x.shape[1])
    return pl.pallas_call(
        body,
        grid=(E // rows,),
        in_specs=[_row_spec(rows, x.shape[1])] + wspecs,
        out_specs=_row_spec(rows, D),
        out_shape=jax.ShapeDtypeStruct((E, D), jnp.float32),
    )(x, *wargs)


def _edge_step(e, ps, qr, sp):
    rows = 2000
    w1e = sp['w1'][0:D]

    def body(e_ref, ps_ref, qr_ref, w1_ref, b1_ref, w2_ref, b2_ref,
             g_ref, beta_ref, o_ref):
        e_blk = e_ref[...]
        extra = ps_ref[...] + qr_ref[...]
        y = _mlp_block(e_blk, w1_ref[...], b1_ref[...], w2_ref[...],
                       b2_ref[...], g_ref[...], beta_ref[...], extra=extra)
        o_ref[...] = e_blk + y

    wargs = [w1e, sp['b1'].reshape(1, -1), sp['w2'], sp['b2'].reshape(1, -1),
             sp['g'].reshape(1, -1), sp['beta'].reshape(1, -1)]
    wspecs = [_full_spec(a.shape) for a in wargs]
    return pl.pallas_call(
        body,
        grid=(E // rows,),
        in_specs=[_row_spec(rows, D)] * 3 + wspecs,
        out_specs=_row_spec(rows, D),
        out_shape=jax.ShapeDtypeStruct((E, D), jnp.float32),
    )(e, ps, qr, *wargs)


def _node_step(v, agg2, sp, nxt):
    rows = 1000
    w1v = sp['w1'][0:D]
    w1a = sp['w1'][D:2 * D]
    with_pq = nxt is not None

    def body(v_ref, a_ref, wv_ref, wa_ref, b1_ref, w2_ref, b2_ref,
             g_ref, beta_ref, *rest):
        if with_pq:
            ws_ref, wr_ref, o_ref, p_ref, q_ref = rest
        else:
            (o_ref,) = rest
        v_blk = v_ref[...]
        agg = a_ref[0] + a_ref[1]
        x1 = _dot(v_blk, wv_ref[...]) + _dot(agg, wa_ref[...]) + b1_ref[...]
        h = jnp.maximum(x1, 0.0)
        y = _dot(h, w2_ref[...]) + b2_ref[...]
        v_new = v_blk + _ln(y, g_ref[...], beta_ref[...])
        o_ref[...] = v_new
        if with_pq:
            p_ref[...] = _dot(v_new, ws_ref[...])
            q_ref[...] = _dot(v_new, wr_ref[...])

    wargs = [w1v, w1a, sp['b1'].reshape(1, -1), sp['w2'],
             sp['b2'].reshape(1, -1), sp['g'].reshape(1, -1),
             sp['beta'].reshape(1, -1)]
    extra_args = []
    if with_pq:
        extra_args = [nxt['w1'][D:2 * D], nxt['w1'][2 * D:3 * D]]
    in_specs = ([_row_spec(rows, D),
                 pl.BlockSpec((2, rows, D), lambda i: (0, i, 0))]
                + [_full_spec(a.shape) for a in wargs]
                + [_full_spec((D, D)) for _ in extra_args])
    n_out = 3 if with_pq else 1
    out_sds = jax.ShapeDtypeStruct((N, D), jnp.float32)
    res = pl.pallas_call(
        body,
        grid=(N // rows,),
        in_specs=in_specs,
        out_specs=[_row_spec(rows, D)] * n_out,
        out_shape=[out_sds] * n_out,
    )(v, agg2, *wargs, *extra_args)
    if with_pq:
        return res
    return res[0], None, None


def _decode(v, dec):
    rows = 1000

    def body(v_ref, w1_ref, b1_ref, w2_ref, b2_ref, o_ref):
        o_ref[...] = _mlp_block(v_ref[...], w1_ref[...], b1_ref[...],
                                w2_ref[...], b2_ref[...])

    wargs = [dec['w1'], dec['b1'].reshape(1, -1), dec['w2'],
             dec['b2'].reshape(1, -1)]
    wspecs = [_full_spec(a.shape) for a in wargs]
    out_cols = dec['w2'].shape[1]
    return pl.pallas_call(
        body,
        grid=(N // rows,),
        in_specs=[_row_spec(rows, D)] + wspecs,
        out_specs=_row_spec(rows, out_cols),
        out_shape=jax.ShapeDtypeStruct((N, out_cols), jnp.float32),
    )(v, *wargs)


@functools.cache
def _sc_mesh():
    return plsc.VectorSubcoreMesh(
        core_axis_name="c", subcore_axis_name="s",
        num_cores=NC, num_subcores=NS)


@functools.cache
def _sc_gather_kernel():
    @functools.partial(
        pl.kernel,
        out_type=[jax.ShapeDtypeStruct((E, D), jnp.float32),
                  jax.ShapeDtypeStruct((E, D), jnp.float32)],
        mesh=_sc_mesh(),
        scratch_types=[
            pltpu.VMEM((C,), jnp.int32),
            pltpu.VMEM((C,), jnp.int32),
            pltpu.VMEM((C, D), jnp.float32),
            pltpu.VMEM((C, D), jnp.float32),
            pltpu.SemaphoreType.DMA,
            pltpu.SemaphoreType.DMA,
        ],
    )
    def gather(p_hbm, q_hbm, s_hbm, r_hbm, ps_hbm, qr_hbm,
               sidx, ridx, bufp, bufq, sem1, sem2):
        cid = lax.axis_index("c")
        sid = lax.axis_index("s")
        wid = sid * NC + cid

        def body(i, carry):
            b = wid + i * NW

            @pl.when(b < NB)
            def _():
                base = b * C
                pltpu.sync_copy(s_hbm.at[pl.ds(base, C)], sidx)
                pltpu.sync_copy(r_hbm.at[pl.ds(base, C)], ridx)
                cp1 = pltpu.async_copy(p_hbm.at[sidx], bufp, sem1)
                cp2 = pltpu.async_copy(q_hbm.at[ridx], bufq, sem2)
                cp1.wait()
                cp2.wait()
                pltpu.sync_copy(bufp, ps_hbm.at[pl.ds(base, C)])
                pltpu.sync_copy(bufq, qr_hbm.at[pl.ds(base, C)])

            return carry

        lax.fori_loop(0, ITERS, body, 0)

    return gather


def _sc_gather(p_tab, q_tab, senders, receivers):
    return _sc_gather_kernel()(p_tab, q_tab, senders, receivers)


_BLK_PER_SC = NB // NC
_SC_ITERS = -(-_BLK_PER_SC // NS)
_WB_ROWS = 80
_WB_BLOCKS = N // _WB_ROWS
_WB_ITERS = -(-_WB_BLOCKS // NS)


@functools.cache
def _sc_scatter_kernel():
    @functools.partial(
        pl.kernel,
        out_type=jax.ShapeDtypeStruct((NC, N, D), jnp.float32),
        mesh=_sc_mesh(),
        scratch_types=[
            pltpu.VMEM((C,), jnp.int32),
            pltpu.VMEM((C, D), jnp.float32),
            pltpu.VMEM_SHARED((N, D), jnp.float32),
            pltpu.VMEM((_WB_ROWS, D), jnp.float32),
        ],
    )
    def scatter(e_hbm, r_hbm, zeros_hbm, out_hbm, ridx, buf, acc, obuf):
        cid = lax.axis_index("c")
        sid = lax.axis_index("s")

        @pl.when(sid == 0)
        def _():
            pltpu.sync_copy(zeros_hbm, acc)

        plsc.subcore_barrier()

        def body(i, carry):
            b_local = sid + i * NS

            @pl.when(b_local < _BLK_PER_SC)
            def _():
                base = cid * (E // NC) + b_local * C
                pltpu.sync_copy(r_hbm.at[pl.ds(base, C)], ridx)
                pltpu.sync_copy(e_hbm.at[pl.ds(base, C)], buf)
                pltpu.sync_copy(buf, acc.at[ridx], add=True)

            return carry

        lax.fori_loop(0, _SC_ITERS, body, 0)
        plsc.subcore_barrier()

        def wb_body(i, carry):
            b = sid + i * NS

            @pl.when(b < _WB_BLOCKS)
            def _():
                row0 = b * _WB_ROWS
                pltpu.sync_copy(acc.at[pl.ds(row0, _WB_ROWS)], obuf)
                pltpu.sync_copy(obuf, out_hbm.at[cid, pl.ds(row0, _WB_ROWS)])

            return carry

        lax.fori_loop(0, _WB_ITERS, wb_body, 0)

    return scatter


def _sc_scatter(e, receivers, zeros):
    return _sc_scatter_kernel()(e, receivers, zeros)


def kernel(node_features, edge_features, senders, receivers, params):
    steps = params['steps']
    v, p_tab, q_tab = _node_encode(
        node_features, params['node_enc'],
        steps[0]['edge']['w1'][D:2 * D], steps[0]['edge']['w1'][2 * D:3 * D])
    e = _edge_encode(edge_features, params['edge_enc'])
    zeros = jnp.zeros((N, D), jnp.float32)
    for i, sp in enumerate(steps):
        ps, qr = _sc_gather(p_tab, q_tab, senders, receivers)
        e = _edge_step(e, ps, qr, sp['edge'])
        agg2 = _sc_scatter(e, receivers, zeros)
        nxt = steps[i + 1]['edge'] if i + 1 < len(steps) else None
        v, p_tab, q_tab = _node_step(v, agg2, sp['node'], nxt)
    return _decode(v, params['decoder'])


# speedup vs baseline: 4.0762x; 1.8823x over previous
import functools

import jax
import jax.numpy as jnp
from jax import lax
from jax.experimental import pallas as pl
from jax.experimental.pallas import tpu as pltpu
from jax.experimental.pallas import tpu_sc as plsc

N = 10000
E = 320000
D = 128

NC = 2
NS = 16
NW = NC * NS

C = 128
NB = E // C
ITERS = -(-NB // NW)

PREC = jax.lax.Precision.DEFAULT


def _ln(y, g, beta):
    mu = jnp.mean(y, axis=-1, keepdims=True)
    var = jnp.mean((y - mu) * (y - mu), axis=-1, keepdims=True)
    return (y - mu) * lax.rsqrt(var + 1e-5) * g + beta


def _dot(a, b):
    return jnp.dot(a, b, preferred_element_type=jnp.float32, precision=PREC)


def _row_spec(rows, cols):
    return pl.BlockSpec((rows, cols), lambda i: (i, 0))


def _full_spec(shape):
    nd = len(shape)
    return pl.BlockSpec(shape, lambda i: (0,) * nd)


def _weight_args(p, din):
    args = [p['w1'], p['b1'].reshape(1, -1), p['w2'], p['b2'].reshape(1, -1)]
    if 'g' in p:
        args += [p['g'].reshape(1, -1), p['beta'].reshape(1, -1)]
    specs = [_full_spec(a.shape) for a in args]
    return args, specs


def _mlp_block(x, w1, b1, w2, b2, g=None, beta=None, extra=None):
    x1 = _dot(x, w1) + b1
    if extra is not None:
        x1 = x1 + extra
    h = jnp.maximum(x1, 0.0)
    y = _dot(h, w2) + b2
    if g is not None:
        y = _ln(y, g, beta)
    return y


def _node_encode(x, enc, w1s, w1r):
    rows = 1000

    def body(x_ref, w1_ref, b1_ref, w2_ref, b2_ref, g_ref, beta_ref,
             ws_ref, wr_ref, v_ref, p_ref, q_ref):
        v = _mlp_block(x_ref[...], w1_ref[...], b1_ref[...], w2_ref[...],
                       b2_ref[...], g_ref[...], beta_ref[...])
        v_ref[...] = v
        p_ref[...] = _dot(v, ws_ref[...])
        q_ref[...] = _dot(v, wr_ref[...])

    wargs, wspecs = _weight_args(enc, x.shape[1])
    out_sds = jax.ShapeDtypeStruct((N, D), jnp.float32)
    return pl.pallas_call(
        body,
        grid=(N // rows,),
        in_specs=[_row_spec(rows, x.shape[1])] + wspecs
        + [_full_spec((D, D)), _full_spec((D, D))],
        out_specs=[_row_spec(rows, D)] * 3,
        out_shape=[out_sds] * 3,
    )(x, *wargs, w1s, w1r)


def _edge_encode(x, enc):
    rows = 2000

    def body(x_ref, w1_ref, b1_ref, w2_ref, b2_ref, g_ref, beta_ref, o_ref):
        o_ref[...] = _mlp_block(x_ref[...], w1_ref[...], b1_ref[...],
                                w2_ref[...], b2_ref[...], g_ref[...],
                                beta_ref[...])

    wargs, wspecs = _weight_args(enc, x.shape[1])
    return pl.pallas_call(
        body,
        grid=(E // rows,),
        in_specs=[_row_spec(rows, x.shape[1])] + wspecs,
        out_specs=_row_spec(rows, D),
        out_shape=jax.ShapeDtypeStruct((E, D), jnp.float32),
    )(x, *wargs)


def _edge_step(e, ps, qr, sp):
    rows = 2000
    w1e = sp['w1'][0:D]

    def body(e_ref, ps_ref, qr_ref, w1_ref, b1_ref, w2_ref, b2_ref,
             g_ref, beta_ref, o_ref):
        e_blk = e_ref[...]
        extra = ps_ref[...] + qr_ref[...]
        y = _mlp_block(e_blk, w1_ref[...], b1_ref[...], w2_ref[...],
                       b2_ref[...], g_ref[...], beta_ref[...], extra=extra)
        o_ref[...] = e_blk + y

    wargs = [w1e, sp['b1'].reshape(1, -1), sp['w2'], sp['b2'].reshape(1, -1),
             sp['g'].reshape(1, -1), sp['beta'].reshape(1, -1)]
    wspecs = [_full_spec(a.shape) for a in wargs]
    return pl.pallas_call(
        body,
        grid=(E // rows,),
        in_specs=[_row_spec(rows, D)] * 3 + wspecs,
        out_specs=_row_spec(rows, D),
        out_shape=jax.ShapeDtypeStruct((E, D), jnp.float32),
    )(e, ps, qr, *wargs)


def _node_step(v, agg2, sp, nxt):
    rows = 1000
    w1v = sp['w1'][0:D]
    w1a = sp['w1'][D:2 * D]
    with_pq = nxt is not None

    def body(v_ref, a_ref, wv_ref, wa_ref, b1_ref, w2_ref, b2_ref,
             g_ref, beta_ref, *rest):
        if with_pq:
            ws_ref, wr_ref, o_ref, p_ref, q_ref = rest
        else:
            (o_ref,) = rest
        v_blk = v_ref[...]
        agg = a_ref[0] + a_ref[1]
        x1 = _dot(v_blk, wv_ref[...]) + _dot(agg, wa_ref[...]) + b1_ref[...]
        h = jnp.maximum(x1, 0.0)
        y = _dot(h, w2_ref[...]) + b2_ref[...]
        v_new = v_blk + _ln(y, g_ref[...], beta_ref[...])
        o_ref[...] = v_new
        if with_pq:
            p_ref[...] = _dot(v_new, ws_ref[...])
            q_ref[...] = _dot(v_new, wr_ref[...])

    wargs = [w1v, w1a, sp['b1'].reshape(1, -1), sp['w2'],
             sp['b2'].reshape(1, -1), sp['g'].reshape(1, -1),
             sp['beta'].reshape(1, -1)]
    extra_args = []
    if with_pq:
        extra_args = [nxt['w1'][D:2 * D], nxt['w1'][2 * D:3 * D]]
    in_specs = ([_row_spec(rows, D),
                 pl.BlockSpec((2, rows, D), lambda i: (0, i, 0))]
                + [_full_spec(a.shape) for a in wargs]
                + [_full_spec((D, D)) for _ in extra_args])
    n_out = 3 if with_pq else 1
    out_sds = jax.ShapeDtypeStruct((N, D), jnp.float32)
    res = pl.pallas_call(
        body,
        grid=(N // rows,),
        in_specs=in_specs,
        out_specs=[_row_spec(rows, D)] * n_out,
        out_shape=[out_sds] * n_out,
    )(v, agg2, *wargs, *extra_args)
    if with_pq:
        return res
    return res[0], None, None


def _decode(v, dec):
    rows = 1000

    def body(v_ref, w1_ref, b1_ref, w2_ref, b2_ref, o_ref):
        o_ref[...] = _mlp_block(v_ref[...], w1_ref[...], b1_ref[...],
                                w2_ref[...], b2_ref[...])

    wargs = [dec['w1'], dec['b1'].reshape(1, -1), dec['w2'],
             dec['b2'].reshape(1, -1)]
    wspecs = [_full_spec(a.shape) for a in wargs]
    out_cols = dec['w2'].shape[1]
    return pl.pallas_call(
        body,
        grid=(N // rows,),
        in_specs=[_row_spec(rows, D)] + wspecs,
        out_specs=_row_spec(rows, out_cols),
        out_shape=jax.ShapeDtypeStruct((N, out_cols), jnp.float32),
    )(v, *wargs)


@functools.cache
def _sc_mesh():
    return plsc.VectorSubcoreMesh(
        core_axis_name="c", subcore_axis_name="s",
        num_cores=NC, num_subcores=NS)


@functools.cache
def _sc_gather_kernel():
    @functools.partial(
        pl.kernel,
        out_type=[jax.ShapeDtypeStruct((E, D), jnp.float32),
                  jax.ShapeDtypeStruct((E, D), jnp.float32)],
        mesh=_sc_mesh(),
        scratch_types=[
            pltpu.VMEM((C,), jnp.int32), pltpu.VMEM((C,), jnp.int32),
            pltpu.VMEM((C,), jnp.int32), pltpu.VMEM((C,), jnp.int32),
            pltpu.VMEM((C, D), jnp.float32), pltpu.VMEM((C, D), jnp.float32),
            pltpu.VMEM((C, D), jnp.float32), pltpu.VMEM((C, D), jnp.float32),
            pltpu.SemaphoreType.DMA, pltpu.SemaphoreType.DMA,
            pltpu.SemaphoreType.DMA, pltpu.SemaphoreType.DMA,
            pltpu.SemaphoreType.DMA, pltpu.SemaphoreType.DMA,
        ],
    )
    def gather(p_hbm, q_hbm, s_hbm, r_hbm, ps_hbm, qr_hbm,
               sidx0, sidx1, ridx0, ridx1, bufp0, bufp1, bufq0, bufq1,
               semi0, semi1, semg0, semg1, semw0, semw1):
        cid = lax.axis_index("c")
        sid = lax.axis_index("s")
        wid = sid * NC + cid
        sidx = (sidx0, sidx1)
        ridx = (ridx0, ridx1)
        bufp = (bufp0, bufp1)
        bufq = (bufq0, bufq1)
        semi = (semi0, semi1)
        semg = (semg0, semg1)
        semw = (semw0, semw1)

        def blk(j):
            return wid + j * NW

        def start_idx(j, p):
            @pl.when(blk(j) < NB)
            def _():
                base = blk(j) * C
                pltpu.async_copy(s_hbm.at[pl.ds(base, C)], sidx[p], semi[p])
                pltpu.async_copy(r_hbm.at[pl.ds(base, C)], ridx[p], semi[p])

        def wait_idx(j, p):
            @pl.when(blk(j) < NB)
            def _():
                pltpu.make_async_copy(
                    s_hbm.at[pl.ds(0, C)], sidx[p], semi[p]).wait()
                pltpu.make_async_copy(
                    r_hbm.at[pl.ds(0, C)], ridx[p], semi[p]).wait()

        def start_gather(j, p):
            @pl.when(blk(j) < NB)
            def _():
                pltpu.async_copy(p_hbm.at[sidx[p]], bufp[p], semg[p])
                pltpu.async_copy(q_hbm.at[ridx[p]], bufq[p], semg[p])

        def wait_gather(j, p):
            @pl.when(blk(j) < NB)
            def _():
                pltpu.make_async_copy(
                    p_hbm.at[sidx[p]], bufp[p], semg[p]).wait()
                pltpu.make_async_copy(
                    q_hbm.at[ridx[p]], bufq[p], semg[p]).wait()

        def start_write(j, p):
            @pl.when(blk(j) < NB)
            def _():
                base = blk(j) * C
                pltpu.async_copy(bufp[p], ps_hbm.at[pl.ds(base, C)], semw[p])
                pltpu.async_copy(bufq[p], qr_hbm.at[pl.ds(base, C)], semw[p])

        def wait_write(j, p, extra_cond):
            @pl.when(jnp.logical_and(extra_cond, blk(j) < NB))
            def _():
                pltpu.make_async_copy(
                    bufp[p], ps_hbm.at[pl.ds(0, C)], semw[p]).wait()
                pltpu.make_async_copy(
                    bufq[p], qr_hbm.at[pl.ds(0, C)], semw[p]).wait()

        start_idx(0, 0)
        start_idx(1, 1)
        wait_idx(0, 0)
        start_gather(0, 0)

        def body(g, carry):
            for s in (0, 1):
                i = 2 * g + s
                p = s
                o = 1 - s
                wait_gather(i, p)
                start_write(i, p)
                wait_idx(i + 1, o)
                wait_write(i - 1, o, i >= 1)
                start_gather(i + 1, o)
                start_idx(i + 2, p)
            return carry

        lax.fori_loop(0, (ITERS + 1) // 2, body, 0)

    return gather


def _sc_gather(p_tab, q_tab, senders, receivers):
    return _sc_gather_kernel()(p_tab, q_tab, senders, receivers)


_BLK_PER_SC = NB // NC
_SC_ITERS = -(-_BLK_PER_SC // NS)
_WB_ROWS = 80
_WB_BLOCKS = N // _WB_ROWS
_WB_ITERS = -(-_WB_BLOCKS // NS)


@functools.cache
def _sc_scatter_kernel():
    @functools.partial(
        pl.kernel,
        out_type=jax.ShapeDtypeStruct((NC, N, D), jnp.float32),
        mesh=_sc_mesh(),
        scratch_types=[
            pltpu.VMEM((C,), jnp.int32), pltpu.VMEM((C,), jnp.int32),
            pltpu.VMEM((C, D), jnp.float32), pltpu.VMEM((C, D), jnp.float32),
            pltpu.VMEM_SHARED((N, D), jnp.float32),
            pltpu.VMEM((_WB_ROWS, D), jnp.float32),
            pltpu.SemaphoreType.DMA, pltpu.SemaphoreType.DMA,
        ],
    )
    def scatter(e_hbm, r_hbm, zeros_hbm, out_hbm, ridx0, ridx1,
                buf0, buf1, acc, obuf, seml0, seml1):
        cid = lax.axis_index("c")
        sid = lax.axis_index("s")
        ridx = (ridx0, ridx1)
        buf = (buf0, buf1)
        seml = (seml0, seml1)

        @pl.when(sid == 0)
        def _():
            pltpu.sync_copy(zeros_hbm, acc)

        plsc.subcore_barrier()

        def base_of(j):
            return cid * (E // NC) + (sid + j * NS) * C

        def in_range(j):
            return (sid + j * NS) < _BLK_PER_SC

        def start_load(j, p):
            @pl.when(in_range(j))
            def _():
                base = base_of(j)
                pltpu.async_copy(r_hbm.at[pl.ds(base, C)], ridx[p], seml[p])
                pltpu.async_copy(e_hbm.at[pl.ds(base, C)], buf[p], seml[p])

        def wait_load(j, p):
            @pl.when(in_range(j))
            def _():
                pltpu.make_async_copy(
                    r_hbm.at[pl.ds(0, C)], ridx[p], seml[p]).wait()
                pltpu.make_async_copy(
                    e_hbm.at[pl.ds(0, C)], buf[p], seml[p]).wait()

        def do_add(j, p):
            @pl.when(in_range(j))
            def _():
                pltpu.sync_copy(buf[p], acc.at[ridx[p]], add=True)

        start_load(0, 0)
        start_load(1, 1)

        def body(g, carry):
            for s in (0, 1):
                i = 2 * g + s
                wait_load(i, s)
                do_add(i, s)
                start_load(i + 2, s)
            return carry

        lax.fori_loop(0, (_SC_ITERS + 1) // 2, body, 0)
        plsc.subcore_barrier()

        def wb_body(i, carry):
            b = sid + i * NS

            @pl.when(b < _WB_BLOCKS)
            def _():
                row0 = b * _WB_ROWS
                pltpu.sync_copy(acc.at[pl.ds(row0, _WB_ROWS)], obuf)
                pltpu.sync_copy(obuf, out_hbm.at[cid, pl.ds(row0, _WB_ROWS)])

            return carry

        lax.fori_loop(0, _WB_ITERS, wb_body, 0)

    return scatter


def _sc_scatter(e, receivers, zeros):
    return _sc_scatter_kernel()(e, receivers, zeros)


def kernel(node_features, edge_features, senders, receivers, params):
    steps = params['steps']
    v, p_tab, q_tab = _node_encode(
        node_features, params['node_enc'],
        steps[0]['edge']['w1'][D:2 * D], steps[0]['edge']['w1'][2 * D:3 * D])
    e = _edge_encode(edge_features, params['edge_enc'])
    zeros = jnp.zeros((N, D), jnp.float32)
    for i, sp in enumerate(steps):
        ps, qr = _sc_gather(p_tab, q_tab, senders, receivers)
        e = _edge_step(e, ps, qr, sp['edge'])
        agg2 = _sc_scatter(e, receivers, zeros)
        nxt = steps[i + 1]['edge'] if i + 1 < len(steps) else None
        v, p_tab, q_tab = _node_step(v, agg2, sp['node'], nxt)
    return _decode(v, params['decoder'])


# speedup vs baseline: 4.2472x; 1.0420x over previous
import functools

import jax
import jax.numpy as jnp
from jax import lax
from jax.experimental import pallas as pl
from jax.experimental.pallas import tpu as pltpu
from jax.experimental.pallas import tpu_sc as plsc

N = 10000
E = 320000
D = 128

NC = 2
NS = 16
NW = NC * NS

C = 128
NB = E // C
ITERS = -(-NB // NW)

PREC = jax.lax.Precision.DEFAULT


def _ln(y, g, beta):
    mu = jnp.mean(y, axis=-1, keepdims=True)
    var = jnp.mean((y - mu) * (y - mu), axis=-1, keepdims=True)
    return (y - mu) * lax.rsqrt(var + 1e-5) * g + beta


def _dot(a, b):
    return jnp.dot(a, b, preferred_element_type=jnp.float32, precision=PREC)


def _row_spec(rows, cols):
    return pl.BlockSpec((rows, cols), lambda i: (i, 0))


def _full_spec(shape):
    nd = len(shape)
    return pl.BlockSpec(shape, lambda i: (0,) * nd)


def _weight_args(p, din):
    args = [p['w1'], p['b1'].reshape(1, -1), p['w2'], p['b2'].reshape(1, -1)]
    if 'g' in p:
        args += [p['g'].reshape(1, -1), p['beta'].reshape(1, -1)]
    specs = [_full_spec(a.shape) for a in args]
    return args, specs


def _mlp_block(x, w1, b1, w2, b2, g=None, beta=None, extra=None):
    x1 = _dot(x, w1) + b1
    if extra is not None:
        x1 = x1 + extra
    h = jnp.maximum(x1, 0.0)
    y = _dot(h, w2) + b2
    if g is not None:
        y = _ln(y, g, beta)
    return y


def _node_encode(x, enc, w1s, w1r):
    rows = 1000

    def body(x_ref, w1_ref, b1_ref, w2_ref, b2_ref, g_ref, beta_ref,
             ws_ref, wr_ref, v_ref, p_ref, q_ref):
        v = _mlp_block(x_ref[...], w1_ref[...], b1_ref[...], w2_ref[...],
                       b2_ref[...], g_ref[...], beta_ref[...])
        v_ref[...] = v
        p_ref[...] = _dot(v, ws_ref[...])
        q_ref[...] = _dot(v, wr_ref[...])

    wargs, wspecs = _weight_args(enc, x.shape[1])
    out_sds = jax.ShapeDtypeStruct((N, D), jnp.float32)
    return pl.pallas_call(
        body,
        grid=(N // rows,),
        in_specs=[_row_spec(rows, x.shape[1])] + wspecs
        + [_full_spec((D, D)), _full_spec((D, D))],
        out_specs=[_row_spec(rows, D)] * 3,
        out_shape=[out_sds] * 3,
    )(x, *wargs, w1s, w1r)


def _edge_encode(x, enc):
    rows = 2000

    def body(x_ref, w1_ref, b1_ref, w2_ref, b2_ref, g_ref, beta_ref, o_ref):
        o_ref[...] = _mlp_block(x_ref[...], w1_ref[...], b1_ref[...],
                                w2_ref[...], b2_ref[...], g_ref[...],
                                beta_ref[...])

    wargs, wspecs = _weight_args(enc, x.shape[1])
    return pl.pallas_call(
        body,
        grid=(E // rows,),
        in_specs=[_row_spec(rows, x.shape[1])] + wspecs,
        out_specs=_row_spec(rows, D),
        out_shape=jax.ShapeDtypeStruct((E, D), jnp.float32),
    )(x, *wargs)


def _edge_step(e, gsum, sp):
    rows = 2000
    w1e = sp['w1'][0:D]

    def body(e_ref, gs_ref, w1_ref, b1_ref, w2_ref, b2_ref,
             g_ref, beta_ref, o_ref):
        e_blk = e_ref[...]
        y = _mlp_block(e_blk, w1_ref[...], b1_ref[...], w2_ref[...],
                       b2_ref[...], g_ref[...], beta_ref[...],
                       extra=gs_ref[...])
        o_ref[...] = e_blk + y

    wargs = [w1e, sp['b1'].reshape(1, -1), sp['w2'], sp['b2'].reshape(1, -1),
             sp['g'].reshape(1, -1), sp['beta'].reshape(1, -1)]
    wspecs = [_full_spec(a.shape) for a in wargs]
    return pl.pallas_call(
        body,
        grid=(E // rows,),
        in_specs=[_row_spec(rows, D)] * 2 + wspecs,
        out_specs=_row_spec(rows, D),
        out_shape=jax.ShapeDtypeStruct((E, D), jnp.float32),
    )(e, gsum, *wargs)


def _node_step(v, agg2, sp, nxt):
    rows = 1000
    w1v = sp['w1'][0:D]
    w1a = sp['w1'][D:2 * D]
    with_pq = nxt is not None

    def body(v_ref, a_ref, wv_ref, wa_ref, b1_ref, w2_ref, b2_ref,
             g_ref, beta_ref, *rest):
        if with_pq:
            ws_ref, wr_ref, o_ref, p_ref, q_ref = rest
        else:
            (o_ref,) = rest
        v_blk = v_ref[...]
        agg = a_ref[0] + a_ref[1]
        x1 = _dot(v_blk, wv_ref[...]) + _dot(agg, wa_ref[...]) + b1_ref[...]
        h = jnp.maximum(x1, 0.0)
        y = _dot(h, w2_ref[...]) + b2_ref[...]
        v_new = v_blk + _ln(y, g_ref[...], beta_ref[...])
        o_ref[...] = v_new
        if with_pq:
            p_ref[...] = _dot(v_new, ws_ref[...])
            q_ref[...] = _dot(v_new, wr_ref[...])

    wargs = [w1v, w1a, sp['b1'].reshape(1, -1), sp['w2'],
             sp['b2'].reshape(1, -1), sp['g'].reshape(1, -1),
             sp['beta'].reshape(1, -1)]
    extra_args = []
    if with_pq:
        extra_args = [nxt['w1'][D:2 * D], nxt['w1'][2 * D:3 * D]]
    in_specs = ([_row_spec(rows, D),
                 pl.BlockSpec((2, rows, D), lambda i: (0, i, 0))]
                + [_full_spec(a.shape) for a in wargs]
                + [_full_spec((D, D)) for _ in extra_args])
    n_out = 3 if with_pq else 1
    out_sds = jax.ShapeDtypeStruct((N, D), jnp.float32)
    res = pl.pallas_call(
        body,
        grid=(N // rows,),
        in_specs=in_specs,
        out_specs=[_row_spec(rows, D)] * n_out,
        out_shape=[out_sds] * n_out,
    )(v, agg2, *wargs, *extra_args)
    if with_pq:
        return res
    return res[0], None, None


def _decode(v, dec):
    rows = 1000

    def body(v_ref, w1_ref, b1_ref, w2_ref, b2_ref, o_ref):
        o_ref[...] = _mlp_block(v_ref[...], w1_ref[...], b1_ref[...],
                                w2_ref[...], b2_ref[...])

    wargs = [dec['w1'], dec['b1'].reshape(1, -1), dec['w2'],
             dec['b2'].reshape(1, -1)]
    wspecs = [_full_spec(a.shape) for a in wargs]
    out_cols = dec['w2'].shape[1]
    return pl.pallas_call(
        body,
        grid=(N // rows,),
        in_specs=[_row_spec(rows, D)] + wspecs,
        out_specs=_row_spec(rows, out_cols),
        out_shape=jax.ShapeDtypeStruct((N, out_cols), jnp.float32),
    )(v, *wargs)


@functools.cache
def _sc_mesh():
    return plsc.VectorSubcoreMesh(
        core_axis_name="c", subcore_axis_name="s",
        num_cores=NC, num_subcores=NS)


@functools.cache
def _sc_gather_kernel():
    @functools.partial(
        pl.kernel,
        out_type=jax.ShapeDtypeStruct((E, D), jnp.float32),
        mesh=_sc_mesh(),
        scratch_types=[
            pltpu.VMEM((C,), jnp.int32), pltpu.VMEM((C,), jnp.int32),
            pltpu.VMEM((C,), jnp.int32), pltpu.VMEM((C,), jnp.int32),
            pltpu.VMEM((C, D), jnp.float32), pltpu.VMEM((C, D), jnp.float32),
            pltpu.VMEM((C, D), jnp.float32), pltpu.VMEM((C, D), jnp.float32),
            pltpu.SemaphoreType.DMA, pltpu.SemaphoreType.DMA,
            pltpu.SemaphoreType.DMA, pltpu.SemaphoreType.DMA,
            pltpu.SemaphoreType.DMA, pltpu.SemaphoreType.DMA,
        ],
    )
    def gather(p_hbm, q_hbm, s_hbm, r_hbm, g_hbm,
               sidx0, sidx1, ridx0, ridx1, bufp0, bufp1, bufq0, bufq1,
               semi0, semi1, semg0, semg1, semw0, semw1):
        cid = lax.axis_index("c")
        sid = lax.axis_index("s")
        wid = sid * NC + cid
        sidx = (sidx0, sidx1)
        ridx = (ridx0, ridx1)
        bufp = (bufp0, bufp1)
        bufq = (bufq0, bufq1)
        semi = (semi0, semi1)
        semg = (semg0, semg1)
        semw = (semw0, semw1)

        def blk(j):
            return wid + j * NW

        def start_idx(j, p):
            @pl.when(blk(j) < NB)
            def _():
                base = blk(j) * C
                pltpu.async_copy(s_hbm.at[pl.ds(base, C)], sidx[p], semi[p])
                pltpu.async_copy(r_hbm.at[pl.ds(base, C)], ridx[p], semi[p])

        def wait_idx(j, p):
            @pl.when(blk(j) < NB)
            def _():
                pltpu.make_async_copy(
                    s_hbm.at[pl.ds(0, C)], sidx[p], semi[p]).wait()
                pltpu.make_async_copy(
                    r_hbm.at[pl.ds(0, C)], ridx[p], semi[p]).wait()

        def start_gather(j, p):
            @pl.when(blk(j) < NB)
            def _():
                pltpu.async_copy(p_hbm.at[sidx[p]], bufp[p], semg[p])
                pltpu.async_copy(q_hbm.at[ridx[p]], bufq[p], semg[p])

        def wait_gather(j, p):
            @pl.when(blk(j) < NB)
            def _():
                pltpu.make_async_copy(
                    p_hbm.at[sidx[p]], bufp[p], semg[p]).wait()
                pltpu.make_async_copy(
                    q_hbm.at[ridx[p]], bufq[p], semg[p]).wait()

        def add_rows(j, p):
            @pl.when(blk(j) < NB)
            def _():
                def row(r, carry):
                    for k in range(D // 16):
                        sl = (r, pl.ds(k * 16, 16))
                        bufp[p][sl] = bufp[p][sl] + bufq[p][sl]
                    return carry

                lax.fori_loop(0, C, row, 0)

        def start_write(j, p):
            @pl.when(blk(j) < NB)
            def _():
                base = blk(j) * C
                pltpu.async_copy(bufp[p], g_hbm.at[pl.ds(base, C)], semw[p])

        def wait_write(j, p, extra_cond):
            @pl.when(jnp.logical_and(extra_cond, blk(j) < NB))
            def _():
                pltpu.make_async_copy(
                    bufp[p], g_hbm.at[pl.ds(0, C)], semw[p]).wait()

        start_idx(0, 0)
        start_idx(1, 1)
        wait_idx(0, 0)
        start_gather(0, 0)

        def body(g, carry):
            for s in (0, 1):
                i = 2 * g + s
                p = s
                o = 1 - s
                wait_gather(i, p)
                add_rows(i, p)
                start_write(i, p)
                wait_idx(i + 1, o)
                wait_write(i - 1, o, i >= 1)
                start_gather(i + 1, o)
                start_idx(i + 2, p)
            return carry

        lax.fori_loop(0, (ITERS + 1) // 2, body, 0)

    return gather


def _sc_gather(p_tab, q_tab, senders, receivers):
    return _sc_gather_kernel()(p_tab, q_tab, senders, receivers)


_BLK_PER_SC = NB // NC
_SC_ITERS = -(-_BLK_PER_SC // NS)
_WB_ROWS = 80
_WB_BLOCKS = N // _WB_ROWS
_WB_ITERS = -(-_WB_BLOCKS // NS)


@functools.cache
def _sc_scatter_kernel():
    @functools.partial(
        pl.kernel,
        out_type=jax.ShapeDtypeStruct((NC, N, D), jnp.float32),
        mesh=_sc_mesh(),
        scratch_types=[
            pltpu.VMEM((C,), jnp.int32), pltpu.VMEM((C,), jnp.int32),
            pltpu.VMEM((C, D), jnp.float32), pltpu.VMEM((C, D), jnp.float32),
            pltpu.VMEM_SHARED((N, D), jnp.float32),
            pltpu.VMEM((_WB_ROWS, D), jnp.float32),
            pltpu.SemaphoreType.DMA, pltpu.SemaphoreType.DMA,
        ],
    )
    def scatter(e_hbm, r_hbm, zeros_hbm, out_hbm, ridx0, ridx1,
                buf0, buf1, acc, obuf, seml0, seml1):
        cid = lax.axis_index("c")
        sid = lax.axis_index("s")
        ridx = (ridx0, ridx1)
        buf = (buf0, buf1)
        seml = (seml0, seml1)

        @pl.when(sid == 0)
        def _():
            pltpu.sync_copy(zeros_hbm, acc)

        plsc.subcore_barrier()

        def base_of(j):
            return cid * (E // NC) + (sid + j * NS) * C

        def in_range(j):
            return (sid + j * NS) < _BLK_PER_SC

        def start_load(j, p):
            @pl.when(in_range(j))
            def _():
                base = base_of(j)
                pltpu.async_copy(r_hbm.at[pl.ds(base, C)], ridx[p], seml[p])
                pltpu.async_copy(e_hbm.at[pl.ds(base, C)], buf[p], seml[p])

        def wait_load(j, p):
            @pl.when(in_range(j))
            def _():
                pltpu.make_async_copy(
                    r_hbm.at[pl.ds(0, C)], ridx[p], seml[p]).wait()
                pltpu.make_async_copy(
                    e_hbm.at[pl.ds(0, C)], buf[p], seml[p]).wait()

        def do_add(j, p):
            @pl.when(in_range(j))
            def _():
                pltpu.sync_copy(buf[p], acc.at[ridx[p]], add=True)

        start_load(0, 0)
        start_load(1, 1)

        def body(g, carry):
            for s in (0, 1):
                i = 2 * g + s
                wait_load(i, s)
                do_add(i, s)
                start_load(i + 2, s)
            return carry

        lax.fori_loop(0, (_SC_ITERS + 1) // 2, body, 0)
        plsc.subcore_barrier()

        def wb_body(i, carry):
            b = sid + i * NS

            @pl.when(b < _WB_BLOCKS)
            def _():
                row0 = b * _WB_ROWS
                pltpu.sync_copy(acc.at[pl.ds(row0, _WB_ROWS)], obuf)
                pltpu.sync_copy(obuf, out_hbm.at[cid, pl.ds(row0, _WB_ROWS)])

            return carry

        lax.fori_loop(0, _WB_ITERS, wb_body, 0)

    return scatter


def _sc_scatter(e, receivers, zeros):
    return _sc_scatter_kernel()(e, receivers, zeros)


def kernel(node_features, edge_features, senders, receivers, params):
    steps = params['steps']
    v, p_tab, q_tab = _node_encode(
        node_features, params['node_enc'],
        steps[0]['edge']['w1'][D:2 * D], steps[0]['edge']['w1'][2 * D:3 * D])
    e = _edge_encode(edge_features, params['edge_enc'])
    zeros = jnp.zeros((N, D), jnp.float32)
    for i, sp in enumerate(steps):
        gsum = _sc_gather(p_tab, q_tab, senders, receivers)
        e = _edge_step(e, gsum, sp['edge'])
        agg2 = _sc_scatter(e, receivers, zeros)
        nxt = steps[i + 1]['edge'] if i + 1 < len(steps) else None
        v, p_tab, q_tab = _node_step(v, agg2, sp['node'], nxt)
    return _decode(v, params['decoder'])


# speedup vs baseline: 4.3437x; 1.0227x over previous
import functools

import jax
import jax.numpy as jnp
from jax import lax
from jax.experimental import pallas as pl
from jax.experimental.pallas import tpu as pltpu
from jax.experimental.pallas import tpu_sc as plsc

N = 10000
E = 320000
D = 128

NC = 2
NS = 16
NW = NC * NS

C = 128
NB = E // C
ITERS = -(-NB // NW)

PREC = jax.lax.Precision.DEFAULT


def _ln(y, g, beta):
    mu = jnp.mean(y, axis=-1, keepdims=True)
    var = jnp.mean((y - mu) * (y - mu), axis=-1, keepdims=True)
    return (y - mu) * lax.rsqrt(var + 1e-5) * g + beta


def _dot(a, b):
    return jnp.dot(a, b, preferred_element_type=jnp.float32, precision=PREC)


def _row_spec(rows, cols):
    return pl.BlockSpec((rows, cols), lambda i: (i, 0))


def _full_spec(shape):
    nd = len(shape)
    return pl.BlockSpec(shape, lambda i: (0,) * nd)


def _weight_args(p, din):
    args = [p['w1'], p['b1'].reshape(1, -1), p['w2'], p['b2'].reshape(1, -1)]
    if 'g' in p:
        args += [p['g'].reshape(1, -1), p['beta'].reshape(1, -1)]
    specs = [_full_spec(a.shape) for a in args]
    return args, specs


def _mlp_block(x, w1, b1, w2, b2, g=None, beta=None, extra=None):
    x1 = _dot(x, w1) + b1
    if extra is not None:
        x1 = x1 + extra
    h = jnp.maximum(x1, 0.0)
    y = _dot(h, w2) + b2
    if g is not None:
        y = _ln(y, g, beta)
    return y


def _node_encode(x, enc, w1s, w1r):
    rows = 1000

    def body(x_ref, w1_ref, b1_ref, w2_ref, b2_ref, g_ref, beta_ref,
             ws_ref, wr_ref, v_ref, p_ref, q_ref):
        v = _mlp_block(x_ref[...], w1_ref[...], b1_ref[...], w2_ref[...],
                       b2_ref[...], g_ref[...], beta_ref[...])
        v_ref[...] = v
        p_ref[...] = _dot(v, ws_ref[...])
        q_ref[...] = _dot(v, wr_ref[...])

    wargs, wspecs = _weight_args(enc, x.shape[1])
    out_sds = jax.ShapeDtypeStruct((N, D), jnp.float32)
    return pl.pallas_call(
        body,
        grid=(N // rows,),
        in_specs=[_row_spec(rows, x.shape[1])] + wspecs
        + [_full_spec((D, D)), _full_spec((D, D))],
        out_specs=[_row_spec(rows, D)] * 3,
        out_shape=[out_sds] * 3,
    )(x, *wargs, w1s, w1r)


def _edge_encode(x, enc):
    rows = 2000

    def body(x_ref, w1_ref, b1_ref, w2_ref, b2_ref, g_ref, beta_ref, o_ref):
        o_ref[...] = _mlp_block(x_ref[...], w1_ref[...], b1_ref[...],
                                w2_ref[...], b2_ref[...], g_ref[...],
                                beta_ref[...])

    wargs, wspecs = _weight_args(enc, x.shape[1])
    return pl.pallas_call(
        body,
        grid=(E // rows,),
        in_specs=[_row_spec(rows, x.shape[1])] + wspecs,
        out_specs=_row_spec(rows, D),
        out_shape=jax.ShapeDtypeStruct((E, D), jnp.float32),
    )(x, *wargs)


def _edge_step(e, gsum, sp):
    rows = 2000
    w1e = sp['w1'][0:D]

    def body(e_ref, gs_ref, w1_ref, b1_ref, w2_ref, b2_ref,
             g_ref, beta_ref, o_ref):
        e_blk = e_ref[...]
        y = _mlp_block(e_blk, w1_ref[...], b1_ref[...], w2_ref[...],
                       b2_ref[...], g_ref[...], beta_ref[...],
                       extra=gs_ref[...])
        o_ref[...] = e_blk + y

    wargs = [w1e, sp['b1'].reshape(1, -1), sp['w2'], sp['b2'].reshape(1, -1),
             sp['g'].reshape(1, -1), sp['beta'].reshape(1, -1)]
    wspecs = [_full_spec(a.shape) for a in wargs]
    return pl.pallas_call(
        body,
        grid=(E // rows,),
        in_specs=[_row_spec(rows, D)] * 2 + wspecs,
        out_specs=_row_spec(rows, D),
        out_shape=jax.ShapeDtypeStruct((E, D), jnp.float32),
    )(e, gsum, *wargs)


def _node_step(v, agg2, sp, nxt):
    rows = 1000
    w1v = sp['w1'][0:D]
    w1a = sp['w1'][D:2 * D]
    with_pq = nxt is not None

    def body(v_ref, a_ref, wv_ref, wa_ref, b1_ref, w2_ref, b2_ref,
             g_ref, beta_ref, *rest):
        if with_pq:
            ws_ref, wr_ref, o_ref, p_ref, q_ref = rest
        else:
            (o_ref,) = rest
        v_blk = v_ref[...]
        agg = a_ref[0] + a_ref[1]
        x1 = _dot(v_blk, wv_ref[...]) + _dot(agg, wa_ref[...]) + b1_ref[...]
        h = jnp.maximum(x1, 0.0)
        y = _dot(h, w2_ref[...]) + b2_ref[...]
        v_new = v_blk + _ln(y, g_ref[...], beta_ref[...])
        o_ref[...] = v_new
        if with_pq:
            p_ref[...] = _dot(v_new, ws_ref[...])
            q_ref[...] = _dot(v_new, wr_ref[...])

    wargs = [w1v, w1a, sp['b1'].reshape(1, -1), sp['w2'],
             sp['b2'].reshape(1, -1), sp['g'].reshape(1, -1),
             sp['beta'].reshape(1, -1)]
    extra_args = []
    if with_pq:
        extra_args = [nxt['w1'][D:2 * D], nxt['w1'][2 * D:3 * D]]
    in_specs = ([_row_spec(rows, D),
                 pl.BlockSpec((2, rows, D), lambda i: (0, i, 0))]
                + [_full_spec(a.shape) for a in wargs]
                + [_full_spec((D, D)) for _ in extra_args])
    n_out = 3 if with_pq else 1
    out_sds = jax.ShapeDtypeStruct((N, D), jnp.float32)
    res = pl.pallas_call(
        body,
        grid=(N // rows,),
        in_specs=in_specs,
        out_specs=[_row_spec(rows, D)] * n_out,
        out_shape=[out_sds] * n_out,
    )(v, agg2, *wargs, *extra_args)
    if with_pq:
        return res
    return res[0], None, None


def _decode(v, dec):
    rows = 1000

    def body(v_ref, w1_ref, b1_ref, w2_ref, b2_ref, o_ref):
        o_ref[...] = _mlp_block(v_ref[...], w1_ref[...], b1_ref[...],
                                w2_ref[...], b2_ref[...])

    wargs = [dec['w1'], dec['b1'].reshape(1, -1), dec['w2'],
             dec['b2'].reshape(1, -1)]
    wspecs = [_full_spec(a.shape) for a in wargs]
    out_cols = dec['w2'].shape[1]
    return pl.pallas_call(
        body,
        grid=(N // rows,),
        in_specs=[_row_spec(rows, D)] + wspecs,
        out_specs=_row_spec(rows, out_cols),
        out_shape=jax.ShapeDtypeStruct((N, out_cols), jnp.float32),
    )(v, *wargs)


@functools.cache
def _sc_mesh():
    return plsc.VectorSubcoreMesh(
        core_axis_name="c", subcore_axis_name="s",
        num_cores=NC, num_subcores=NS)


@functools.cache
def _sc_gather_kernel():
    @functools.partial(
        pl.kernel,
        out_type=jax.ShapeDtypeStruct((E, D), jnp.float32),
        mesh=_sc_mesh(),
        scratch_types=[
            pltpu.VMEM((C,), jnp.int32), pltpu.VMEM((C,), jnp.int32),
            pltpu.VMEM((C,), jnp.int32), pltpu.VMEM((C,), jnp.int32),
            pltpu.VMEM((C, D), jnp.float32),
            pltpu.VMEM((C, D), jnp.float32), pltpu.VMEM((C, D), jnp.float32),
            pltpu.VMEM_SHARED((N, D), jnp.float32),
            pltpu.SemaphoreType.DMA, pltpu.SemaphoreType.DMA,
            pltpu.SemaphoreType.DMA, pltpu.SemaphoreType.DMA,
            pltpu.SemaphoreType.DMA, pltpu.SemaphoreType.DMA,
        ],
    )
    def gather(p_hbm, q_hbm, s_hbm, r_hbm, g_hbm,
               sidx0, sidx1, ridx0, ridx1, bufp, bufq0, bufq1,
               ptab_sh, semi0, semi1, semg0, semg1, semw0, semw1):
        cid = lax.axis_index("c")
        sid = lax.axis_index("s")
        wid = sid * NC + cid
        sidx = (sidx0, sidx1)
        ridx = (ridx0, ridx1)
        bufq = (bufq0, bufq1)
        semi = (semi0, semi1)
        semg = (semg0, semg1)
        semw = (semw0, semw1)

        def blk(j):
            return wid + j * NW

        def start_idx(j, p):
            @pl.when(blk(j) < NB)
            def _():
                base = blk(j) * C
                pltpu.async_copy(s_hbm.at[pl.ds(base, C)], sidx[p], semi[p])
                pltpu.async_copy(r_hbm.at[pl.ds(base, C)], ridx[p], semi[p])

        def wait_idx(j, p):
            @pl.when(blk(j) < NB)
            def _():
                pltpu.make_async_copy(
                    s_hbm.at[pl.ds(0, C)], sidx[p], semi[p]).wait()
                pltpu.make_async_copy(
                    r_hbm.at[pl.ds(0, C)], ridx[p], semi[p]).wait()

        def start_gather(j, p):
            @pl.when(blk(j) < NB)
            def _():
                pltpu.async_copy(q_hbm.at[ridx[p]], bufq[p], semg[p])

        def wait_gather(j, p):
            @pl.when(blk(j) < NB)
            def _():
                pltpu.make_async_copy(
                    q_hbm.at[ridx[p]], bufq[p], semg[p]).wait()

        def add_p_rows(j, p):
            @pl.when(blk(j) < NB)
            def _():
                pltpu.async_copy(ptab_sh.at[sidx[p]], bufp, semg[p]).wait()

                def row(r, carry):
                    for k in range(D // 16):
                        sl = (r, pl.ds(k * 16, 16))
                        bufq[p][sl] = bufq[p][sl] + bufp[sl]
                    return carry

                lax.fori_loop(0, C, row, 0)

        def start_write(j, p):
            @pl.when(blk(j) < NB)
            def _():
                base = blk(j) * C
                pltpu.async_copy(bufq[p], g_hbm.at[pl.ds(base, C)], semw[p])

        def wait_write(j, p, extra_cond):
            @pl.when(jnp.logical_and(extra_cond, blk(j) < NB))
            def _():
                pltpu.make_async_copy(
                    bufq[p], g_hbm.at[pl.ds(0, C)], semw[p]).wait()

        @pl.when(sid == 0)
        def _():
            pltpu.sync_copy(p_hbm, ptab_sh)

        plsc.subcore_barrier()

        start_idx(0, 0)
        start_idx(1, 1)
        wait_idx(0, 0)
        start_gather(0, 0)

        def body(g, carry):
            for s in (0, 1):
                i = 2 * g + s
                p = s
                o = 1 - s
                wait_gather(i, p)
                add_p_rows(i, p)
                start_write(i, p)
                wait_idx(i + 1, o)
                wait_write(i - 1, o, i >= 1)
                start_gather(i + 1, o)
                start_idx(i + 2, p)
            return carry

        lax.fori_loop(0, (ITERS + 1) // 2, body, 0)

    return gather


def _sc_gather(p_tab, q_tab, senders, receivers):
    return _sc_gather_kernel()(p_tab, q_tab, senders, receivers)


_BLK_PER_SC = NB // NC
_SC_ITERS = -(-_BLK_PER_SC // NS)
_WB_ROWS = 80
_WB_BLOCKS = N // _WB_ROWS
_WB_ITERS = -(-_WB_BLOCKS // NS)


@functools.cache
def _sc_scatter_kernel():
    @functools.partial(
        pl.kernel,
        out_type=jax.ShapeDtypeStruct((NC, N, D), jnp.float32),
        mesh=_sc_mesh(),
        scratch_types=[
            pltpu.VMEM((C,), jnp.int32), pltpu.VMEM((C,), jnp.int32),
            pltpu.VMEM((C, D), jnp.float32), pltpu.VMEM((C, D), jnp.float32),
            pltpu.VMEM_SHARED((N, D), jnp.float32),
            pltpu.VMEM((_WB_ROWS, D), jnp.float32),
            pltpu.SemaphoreType.DMA, pltpu.SemaphoreType.DMA,
        ],
    )
    def scatter(e_hbm, r_hbm, zeros_hbm, out_hbm, ridx0, ridx1,
                buf0, buf1, acc, obuf, seml0, seml1):
        cid = lax.axis_index("c")
        sid = lax.axis_index("s")
        ridx = (ridx0, ridx1)
        buf = (buf0, buf1)
        seml = (seml0, seml1)

        @pl.when(sid == 0)
        def _():
            pltpu.sync_copy(zeros_hbm, acc)

        plsc.subcore_barrier()

        def base_of(j):
            return cid * (E // NC) + (sid + j * NS) * C

        def in_range(j):
            return (sid + j * NS) < _BLK_PER_SC

        def start_load(j, p):
            @pl.when(in_range(j))
            def _():
                base = base_of(j)
                pltpu.async_copy(r_hbm.at[pl.ds(base, C)], ridx[p], seml[p])
                pltpu.async_copy(e_hbm.at[pl.ds(base, C)], buf[p], seml[p])

        def wait_load(j, p):
            @pl.when(in_range(j))
            def _():
                pltpu.make_async_copy(
                    r_hbm.at[pl.ds(0, C)], ridx[p], seml[p]).wait()
                pltpu.make_async_copy(
                    e_hbm.at[pl.ds(0, C)], buf[p], seml[p]).wait()

        def do_add(j, p):
            @pl.when(in_range(j))
            def _():
                pltpu.sync_copy(buf[p], acc.at[ridx[p]], add=True)

        start_load(0, 0)
        start_load(1, 1)

        def body(g, carry):
            for s in (0, 1):
                i = 2 * g + s
                wait_load(i, s)
                do_add(i, s)
                start_load(i + 2, s)
            return carry

        lax.fori_loop(0, (_SC_ITERS + 1) // 2, body, 0)
        plsc.subcore_barrier()

        def wb_body(i, carry):
            b = sid + i * NS

            @pl.when(b < _WB_BLOCKS)
            def _():
                row0 = b * _WB_ROWS
                pltpu.sync_copy(acc.at[pl.ds(row0, _WB_ROWS)], obuf)
                pltpu.sync_copy(obuf, out_hbm.at[cid, pl.ds(row0, _WB_ROWS)])

            return carry

        lax.fori_loop(0, _WB_ITERS, wb_body, 0)

    return scatter


def _sc_scatter(e, receivers, zeros):
    return _sc_scatter_kernel()(e, receivers, zeros)


def kernel(node_features, edge_features, senders, receivers, params):
    steps = params['steps']
    v, p_tab, q_tab = _node_encode(
        node_features, params['node_enc'],
        steps[0]['edge']['w1'][D:2 * D], steps[0]['edge']['w1'][2 * D:3 * D])
    e = _edge_encode(edge_features, params['edge_enc'])
    zeros = jnp.zeros((N, D), jnp.float32)
    for i, sp in enumerate(steps):
        gsum = _sc_gather(p_tab, q_tab, senders, receivers)
        e = _edge_step(e, gsum, sp['edge'])
        agg2 = _sc_scatter(e, receivers, zeros)
        nxt = steps[i + 1]['edge'] if i + 1 < len(steps) else None
        v, p_tab, q_tab = _node_step(v, agg2, sp['node'], nxt)
    return _decode(v, params['decoder'])


# speedup vs baseline: 4.3857x; 1.0097x over previous
import functools

import jax
import jax.numpy as jnp
from jax import lax
from jax.experimental import pallas as pl
from jax.experimental.pallas import tpu as pltpu
from jax.experimental.pallas import tpu_sc as plsc

N = 10000
E = 320000
D = 128

NC = 2
NS = 16
NW = NC * NS

C = 128
NB = E // C
ITERS = -(-NB // NW)

PREC = jax.lax.Precision.DEFAULT


def _ln(y, g, beta):
    mu = jnp.mean(y, axis=-1, keepdims=True)
    var = jnp.mean((y - mu) * (y - mu), axis=-1, keepdims=True)
    return (y - mu) * lax.rsqrt(var + 1e-5) * g + beta


def _dot(a, b):
    return jnp.dot(a, b, preferred_element_type=jnp.float32, precision=PREC)


def _row_spec(rows, cols):
    return pl.BlockSpec((rows, cols), lambda i: (i, 0))


def _full_spec(shape):
    nd = len(shape)
    return pl.BlockSpec(shape, lambda i: (0,) * nd)


def _weight_args(p, din):
    args = [p['w1'], p['b1'].reshape(1, -1), p['w2'], p['b2'].reshape(1, -1)]
    if 'g' in p:
        args += [p['g'].reshape(1, -1), p['beta'].reshape(1, -1)]
    specs = [_full_spec(a.shape) for a in args]
    return args, specs


def _mlp_block(x, w1, b1, w2, b2, g=None, beta=None, extra=None):
    x1 = _dot(x, w1) + b1
    if extra is not None:
        x1 = x1 + extra
    h = jnp.maximum(x1, 0.0)
    y = _dot(h, w2) + b2
    if g is not None:
        y = _ln(y, g, beta)
    return y


def _node_encode(x, enc, w1s, w1r):
    rows = 1000

    def body(x_ref, w1_ref, b1_ref, w2_ref, b2_ref, g_ref, beta_ref,
             ws_ref, wr_ref, v_ref, p_ref, q_ref):
        v = _mlp_block(x_ref[...], w1_ref[...], b1_ref[...], w2_ref[...],
                       b2_ref[...], g_ref[...], beta_ref[...])
        v_ref[...] = v
        p_ref[...] = _dot(v, ws_ref[...])
        q_ref[...] = _dot(v, wr_ref[...])

    wargs, wspecs = _weight_args(enc, x.shape[1])
    out_sds = jax.ShapeDtypeStruct((N, D), jnp.float32)
    return pl.pallas_call(
        body,
        grid=(N // rows,),
        in_specs=[_row_spec(rows, x.shape[1])] + wspecs
        + [_full_spec((D, D)), _full_spec((D, D))],
        out_specs=[_row_spec(rows, D)] * 3,
        out_shape=[out_sds] * 3,
    )(x, *wargs, w1s, w1r)


def _edge_encode(x, enc):
    rows = 2000

    def body(x_ref, w1_ref, b1_ref, w2_ref, b2_ref, g_ref, beta_ref, o_ref):
        o_ref[...] = _mlp_block(x_ref[...], w1_ref[...], b1_ref[...],
                                w2_ref[...], b2_ref[...], g_ref[...],
                                beta_ref[...])

    wargs, wspecs = _weight_args(enc, x.shape[1])
    return pl.pallas_call(
        body,
        grid=(E // rows,),
        in_specs=[_row_spec(rows, x.shape[1])] + wspecs,
        out_specs=_row_spec(rows, D),
        out_shape=jax.ShapeDtypeStruct((E, D), jnp.float32),
    )(x, *wargs)


def _edge_step(e, gsum, sp, enc=None, raw=None):
    rows = 2000
    w1e = sp['w1'][0:D]
    fused_enc = enc is not None

    def body(*refs):
        if fused_enc:
            (x_ref, gs_ref, ew1, eb1, ew2, eb2, eg, ebeta,
             w1_ref, b1_ref, w2_ref, b2_ref, g_ref, beta_ref, o_ref) = refs
            e_blk = _mlp_block(x_ref[...], ew1[...], eb1[...], ew2[...],
                               eb2[...], eg[...], ebeta[...])
        else:
            (e_ref, gs_ref, w1_ref, b1_ref, w2_ref, b2_ref,
             g_ref, beta_ref, o_ref) = refs
            e_blk = e_ref[...]
        y = _mlp_block(e_blk, w1_ref[...], b1_ref[...], w2_ref[...],
                       b2_ref[...], g_ref[...], beta_ref[...],
                       extra=gs_ref[...])
        o_ref[...] = e_blk + y

    wargs = [w1e, sp['b1'].reshape(1, -1), sp['w2'], sp['b2'].reshape(1, -1),
             sp['g'].reshape(1, -1), sp['beta'].reshape(1, -1)]
    if fused_enc:
        eargs, especs = _weight_args(enc, raw.shape[1])
        first = raw
        in_specs = ([_row_spec(rows, raw.shape[1]), _row_spec(rows, D)]
                    + especs + [_full_spec(a.shape) for a in wargs])
        args = [raw, gsum] + eargs + wargs
    else:
        in_specs = ([_row_spec(rows, D)] * 2
                    + [_full_spec(a.shape) for a in wargs])
        args = [e, gsum] + wargs
    return pl.pallas_call(
        body,
        grid=(E // rows,),
        in_specs=in_specs,
        out_specs=_row_spec(rows, D),
        out_shape=jax.ShapeDtypeStruct((E, D), jnp.float32),
    )(*args)


def _node_step(v, agg2, sp, nxt):
    rows = 1000
    w1v = sp['w1'][0:D]
    w1a = sp['w1'][D:2 * D]
    with_pq = nxt is not None

    def body(v_ref, a_ref, wv_ref, wa_ref, b1_ref, w2_ref, b2_ref,
             g_ref, beta_ref, *rest):
        if with_pq:
            ws_ref, wr_ref, o_ref, p_ref, q_ref = rest
        else:
            (o_ref,) = rest
        v_blk = v_ref[...]
        agg = a_ref[0] + a_ref[1]
        x1 = _dot(v_blk, wv_ref[...]) + _dot(agg, wa_ref[...]) + b1_ref[...]
        h = jnp.maximum(x1, 0.0)
        y = _dot(h, w2_ref[...]) + b2_ref[...]
        v_new = v_blk + _ln(y, g_ref[...], beta_ref[...])
        o_ref[...] = v_new
        if with_pq:
            p_ref[...] = _dot(v_new, ws_ref[...])
            q_ref[...] = _dot(v_new, wr_ref[...])

    wargs = [w1v, w1a, sp['b1'].reshape(1, -1), sp['w2'],
             sp['b2'].reshape(1, -1), sp['g'].reshape(1, -1),
             sp['beta'].reshape(1, -1)]
    extra_args = []
    if with_pq:
        extra_args = [nxt['w1'][D:2 * D], nxt['w1'][2 * D:3 * D]]
    in_specs = ([_row_spec(rows, D),
                 pl.BlockSpec((2, rows, D), lambda i: (0, i, 0))]
                + [_full_spec(a.shape) for a in wargs]
                + [_full_spec((D, D)) for _ in extra_args])
    n_out = 3 if with_pq else 1
    out_sds = jax.ShapeDtypeStruct((N, D), jnp.float32)
    res = pl.pallas_call(
        body,
        grid=(N // rows,),
        in_specs=in_specs,
        out_specs=[_row_spec(rows, D)] * n_out,
        out_shape=[out_sds] * n_out,
    )(v, agg2, *wargs, *extra_args)
    if with_pq:
        return res
    return res[0], None, None


def _decode(v, dec):
    rows = 1000

    def body(v_ref, w1_ref, b1_ref, w2_ref, b2_ref, o_ref):
        o_ref[...] = _mlp_block(v_ref[...], w1_ref[...], b1_ref[...],
                                w2_ref[...], b2_ref[...])

    wargs = [dec['w1'], dec['b1'].reshape(1, -1), dec['w2'],
             dec['b2'].reshape(1, -1)]
    wspecs = [_full_spec(a.shape) for a in wargs]
    out_cols = dec['w2'].shape[1]
    return pl.pallas_call(
        body,
        grid=(N // rows,),
        in_specs=[_row_spec(rows, D)] + wspecs,
        out_specs=_row_spec(rows, out_cols),
        out_shape=jax.ShapeDtypeStruct((N, out_cols), jnp.float32),
    )(v, *wargs)


@functools.cache
def _sc_mesh():
    return plsc.VectorSubcoreMesh(
        core_axis_name="c", subcore_axis_name="s",
        num_cores=NC, num_subcores=NS)


@functools.cache
def _sc_gather_kernel():
    @functools.partial(
        pl.kernel,
        out_type=jax.ShapeDtypeStruct((E, D), jnp.float32),
        mesh=_sc_mesh(),
        scratch_types=[
            pltpu.VMEM((C,), jnp.int32), pltpu.VMEM((C,), jnp.int32),
            pltpu.VMEM((C,), jnp.int32), pltpu.VMEM((C,), jnp.int32),
            pltpu.VMEM((C, D), jnp.float32),
            pltpu.VMEM((C, D), jnp.float32), pltpu.VMEM((C, D), jnp.float32),
            pltpu.VMEM_SHARED((N, D), jnp.float32),
            pltpu.SemaphoreType.DMA, pltpu.SemaphoreType.DMA,
            pltpu.SemaphoreType.DMA, pltpu.SemaphoreType.DMA,
            pltpu.SemaphoreType.DMA, pltpu.SemaphoreType.DMA,
        ],
    )
    def gather(p_hbm, q_hbm, s_hbm, r_hbm, g_hbm,
               sidx0, sidx1, ridx0, ridx1, bufp, bufq0, bufq1,
               ptab_sh, semi0, semi1, semg0, semg1, semw0, semw1):
        cid = lax.axis_index("c")
        sid = lax.axis_index("s")
        wid = sid * NC + cid
        sidx = (sidx0, sidx1)
        ridx = (ridx0, ridx1)
        bufq = (bufq0, bufq1)
        semi = (semi0, semi1)
        semg = (semg0, semg1)
        semw = (semw0, semw1)

        def blk(j):
            return wid + j * NW

        def start_idx(j, p):
            @pl.when(blk(j) < NB)
            def _():
                base = blk(j) * C
                pltpu.async_copy(s_hbm.at[pl.ds(base, C)], sidx[p], semi[p])
                pltpu.async_copy(r_hbm.at[pl.ds(base, C)], ridx[p], semi[p])

        def wait_idx(j, p):
            @pl.when(blk(j) < NB)
            def _():
                pltpu.make_async_copy(
                    s_hbm.at[pl.ds(0, C)], sidx[p], semi[p]).wait()
                pltpu.make_async_copy(
                    r_hbm.at[pl.ds(0, C)], ridx[p], semi[p]).wait()

        def start_gather(j, p):
            @pl.when(blk(j) < NB)
            def _():
                pltpu.async_copy(q_hbm.at[ridx[p]], bufq[p], semg[p])

        def wait_gather(j, p):
            @pl.when(blk(j) < NB)
            def _():
                pltpu.make_async_copy(
                    q_hbm.at[ridx[p]], bufq[p], semg[p]).wait()

        def add_p_rows(j, p):
            @pl.when(blk(j) < NB)
            def _():
                pltpu.async_copy(ptab_sh.at[sidx[p]], bufp, semg[p]).wait()

                def row(r, carry):
                    for k in range(D // 16):
                        sl = (r, pl.ds(k * 16, 16))
                        bufq[p][sl] = bufq[p][sl] + bufp[sl]
                    return carry

                lax.fori_loop(0, C, row, 0)

        def start_write(j, p):
            @pl.when(blk(j) < NB)
            def _():
                base = blk(j) * C
                pltpu.async_copy(bufq[p], g_hbm.at[pl.ds(base, C)], semw[p])

        def wait_write(j, p, extra_cond):
            @pl.when(jnp.logical_and(extra_cond, blk(j) < NB))
            def _():
                pltpu.make_async_copy(
                    bufq[p], g_hbm.at[pl.ds(0, C)], semw[p]).wait()

        @pl.when(sid == 0)
        def _():
            pltpu.sync_copy(p_hbm, ptab_sh)

        plsc.subcore_barrier()

        start_idx(0, 0)
        start_idx(1, 1)
        wait_idx(0, 0)
        start_gather(0, 0)

        def body(g, carry):
            for s in (0, 1):
                i = 2 * g + s
                p = s
                o = 1 - s
                wait_gather(i, p)
                add_p_rows(i, p)
                start_write(i, p)
                wait_idx(i + 1, o)
                wait_write(i - 1, o, i >= 1)
                start_gather(i + 1, o)
                start_idx(i + 2, p)
            return carry

        lax.fori_loop(0, (ITERS + 1) // 2, body, 0)

    return gather


def _sc_gather(p_tab, q_tab, senders, receivers):
    return _sc_gather_kernel()(p_tab, q_tab, senders, receivers)


_BLK_PER_SC = NB // NC
_SC_ITERS = -(-_BLK_PER_SC // NS)
_WB_ROWS = 80
_WB_BLOCKS = N // _WB_ROWS
_WB_ITERS = -(-_WB_BLOCKS // NS)


@functools.cache
def _sc_scatter_kernel():
    @functools.partial(
        pl.kernel,
        out_type=jax.ShapeDtypeStruct((NC, N, D), jnp.float32),
        mesh=_sc_mesh(),
        scratch_types=[
            pltpu.VMEM((C,), jnp.int32), pltpu.VMEM((C,), jnp.int32),
            pltpu.VMEM((C, D), jnp.float32), pltpu.VMEM((C, D), jnp.float32),
            pltpu.VMEM_SHARED((N, D), jnp.float32),
            pltpu.VMEM((_WB_ROWS, D), jnp.float32),
            pltpu.SemaphoreType.DMA, pltpu.SemaphoreType.DMA,
        ],
    )
    def scatter(e_hbm, r_hbm, zeros_hbm, out_hbm, ridx0, ridx1,
                buf0, buf1, acc, obuf, seml0, seml1):
        cid = lax.axis_index("c")
        sid = lax.axis_index("s")
        ridx = (ridx0, ridx1)
        buf = (buf0, buf1)
        seml = (seml0, seml1)

        @pl.when(sid == 0)
        def _():
            pltpu.sync_copy(zeros_hbm, acc)

        plsc.subcore_barrier()

        def base_of(j):
            return cid * (E // NC) + (sid + j * NS) * C

        def in_range(j):
            return (sid + j * NS) < _BLK_PER_SC

        def start_load(j, p):
            @pl.when(in_range(j))
            def _():
                base = base_of(j)
                pltpu.async_copy(r_hbm.at[pl.ds(base, C)], ridx[p], seml[p])
                pltpu.async_copy(e_hbm.at[pl.ds(base, C)], buf[p], seml[p])

        def wait_load(j, p):
            @pl.when(in_range(j))
            def _():
                pltpu.make_async_copy(
                    r_hbm.at[pl.ds(0, C)], ridx[p], seml[p]).wait()
                pltpu.make_async_copy(
                    e_hbm.at[pl.ds(0, C)], buf[p], seml[p]).wait()

        def do_add(j, p):
            @pl.when(in_range(j))
            def _():
                pltpu.sync_copy(buf[p], acc.at[ridx[p]], add=True)

        start_load(0, 0)
        start_load(1, 1)

        def body(g, carry):
            for s in (0, 1):
                i = 2 * g + s
                wait_load(i, s)
                do_add(i, s)
                start_load(i + 2, s)
            return carry

        lax.fori_loop(0, (_SC_ITERS + 1) // 2, body, 0)
        plsc.subcore_barrier()

        def wb_body(i, carry):
            b = sid + i * NS

            @pl.when(b < _WB_BLOCKS)
            def _():
                row0 = b * _WB_ROWS
                pltpu.sync_copy(acc.at[pl.ds(row0, _WB_ROWS)], obuf)
                pltpu.sync_copy(obuf, out_hbm.at[cid, pl.ds(row0, _WB_ROWS)])

            return carry

        lax.fori_loop(0, _WB_ITERS, wb_body, 0)

    return scatter


def _sc_scatter(e, receivers, zeros):
    return _sc_scatter_kernel()(e, receivers, zeros)


def kernel(node_features, edge_features, senders, receivers, params):
    steps = params['steps']
    v, p_tab, q_tab = _node_encode(
        node_features, params['node_enc'],
        steps[0]['edge']['w1'][D:2 * D], steps[0]['edge']['w1'][2 * D:3 * D])
    zeros = jnp.zeros((N, D), jnp.float32)
    e = None
    for i, sp in enumerate(steps):
        gsum = _sc_gather(p_tab, q_tab, senders, receivers)
        if i == 0:
            e = _edge_step(None, gsum, sp['edge'],
                           enc=params['edge_enc'], raw=edge_features)
        else:
            e = _edge_step(e, gsum, sp['edge'])
        agg2 = _sc_scatter(e, receivers, zeros)
        nxt = steps[i + 1]['edge'] if i + 1 < len(steps) else None
        v, p_tab, q_tab = _node_step(v, agg2, sp['node'], nxt)
    return _decode(v, params['decoder'])


# speedup vs baseline: 5.1450x; 1.1731x over previous
import functools

import jax
import jax.numpy as jnp
from jax import lax
from jax.experimental import pallas as pl
from jax.experimental.pallas import tpu as pltpu
from jax.experimental.pallas import tpu_sc as plsc

N = 10000
E = 320000
D = 128

NC = 2
NS = 16
NW = NC * NS

C = 128
NB = E // C
ITERS = -(-NB // NW)

PREC = jax.lax.Precision.DEFAULT


def _ln(y, g, beta):
    mu = jnp.mean(y, axis=-1, keepdims=True)
    var = jnp.mean((y - mu) * (y - mu), axis=-1, keepdims=True)
    return (y - mu) * lax.rsqrt(var + 1e-5) * g + beta


def _dot(a, b):
    return jnp.dot(a, b, preferred_element_type=jnp.float32, precision=PREC)


def _row_spec(rows, cols):
    return pl.BlockSpec((rows, cols), lambda i: (i, 0))


def _full_spec(shape):
    nd = len(shape)
    return pl.BlockSpec(shape, lambda i: (0,) * nd)


def _weight_args(p, din):
    args = [p['w1'], p['b1'].reshape(1, -1), p['w2'], p['b2'].reshape(1, -1)]
    if 'g' in p:
        args += [p['g'].reshape(1, -1), p['beta'].reshape(1, -1)]
    specs = [_full_spec(a.shape) for a in args]
    return args, specs


def _mlp_block(x, w1, b1, w2, b2, g=None, beta=None, extra=None):
    x1 = _dot(x, w1) + b1
    if extra is not None:
        x1 = x1 + extra
    h = jnp.maximum(x1, 0.0)
    y = _dot(h, w2) + b2
    if g is not None:
        y = _ln(y, g, beta)
    return y


def _node_encode(x, enc, w1s, w1r):
    rows = 1000

    def body(x_ref, w1_ref, b1_ref, w2_ref, b2_ref, g_ref, beta_ref,
             ws_ref, wr_ref, v_ref, p_ref, q_ref):
        v = _mlp_block(x_ref[...], w1_ref[...], b1_ref[...], w2_ref[...],
                       b2_ref[...], g_ref[...], beta_ref[...])
        v_ref[...] = v
        p_ref[...] = _dot(v, ws_ref[...])
        q_ref[...] = _dot(v, wr_ref[...])

    wargs, wspecs = _weight_args(enc, x.shape[1])
    out_sds = jax.ShapeDtypeStruct((N, D), jnp.float32)
    return pl.pallas_call(
        body,
        grid=(N // rows,),
        in_specs=[_row_spec(rows, x.shape[1])] + wspecs
        + [_full_spec((D, D)), _full_spec((D, D))],
        out_specs=[_row_spec(rows, D)] * 3,
        out_shape=[out_sds] * 3,
    )(x, *wargs, w1s, w1r)


def _edge_encode(x, enc):
    rows = 2000

    def body(x_ref, w1_ref, b1_ref, w2_ref, b2_ref, g_ref, beta_ref, o_ref):
        o_ref[...] = _mlp_block(x_ref[...], w1_ref[...], b1_ref[...],
                                w2_ref[...], b2_ref[...], g_ref[...],
                                beta_ref[...])

    wargs, wspecs = _weight_args(enc, x.shape[1])
    return pl.pallas_call(
        body,
        grid=(E // rows,),
        in_specs=[_row_spec(rows, x.shape[1])] + wspecs,
        out_specs=_row_spec(rows, D),
        out_shape=jax.ShapeDtypeStruct((E, D), jnp.float32),
    )(x, *wargs)


def _edge_step(e, gsum, sp, enc=None, raw=None):
    rows = 2000
    w1e = sp['w1'][0:D]
    fused_enc = enc is not None

    def body(*refs):
        if fused_enc:
            (x_ref, gs_ref, ew1, eb1, ew2, eb2, eg, ebeta,
             w1_ref, b1_ref, w2_ref, b2_ref, g_ref, beta_ref, o_ref) = refs
            e_blk = _mlp_block(x_ref[...], ew1[...], eb1[...], ew2[...],
                               eb2[...], eg[...], ebeta[...])
        else:
            (e_ref, gs_ref, w1_ref, b1_ref, w2_ref, b2_ref,
             g_ref, beta_ref, o_ref) = refs
            e_blk = e_ref[...]
        y = _mlp_block(e_blk, w1_ref[...], b1_ref[...], w2_ref[...],
                       b2_ref[...], g_ref[...], beta_ref[...],
                       extra=gs_ref[...])
        o_ref[...] = e_blk + y

    wargs = [w1e, sp['b1'].reshape(1, -1), sp['w2'], sp['b2'].reshape(1, -1),
             sp['g'].reshape(1, -1), sp['beta'].reshape(1, -1)]
    if fused_enc:
        eargs, especs = _weight_args(enc, raw.shape[1])
        first = raw
        in_specs = ([_row_spec(rows, raw.shape[1]), _row_spec(rows, D)]
                    + especs + [_full_spec(a.shape) for a in wargs])
        args = [raw, gsum] + eargs + wargs
    else:
        in_specs = ([_row_spec(rows, D)] * 2
                    + [_full_spec(a.shape) for a in wargs])
        args = [e, gsum] + wargs
    m = args[0].shape[0]
    return pl.pallas_call(
        body,
        grid=(m // rows,),
        in_specs=in_specs,
        out_specs=_row_spec(rows, D),
        out_shape=jax.ShapeDtypeStruct((m, D), jnp.float32),
    )(*args)


def _node_step(v, agg2, agg2b, sp, nxt):
    rows = 1000
    w1v = sp['w1'][0:D]
    w1a = sp['w1'][D:2 * D]
    with_pq = nxt is not None

    def body(v_ref, a_ref, a2_ref, wv_ref, wa_ref, b1_ref, w2_ref, b2_ref,
             g_ref, beta_ref, *rest):
        if with_pq:
            ws_ref, wr_ref, o_ref, p_ref, q_ref = rest
        else:
            (o_ref,) = rest
        v_blk = v_ref[...]
        agg = (a_ref[0] + a_ref[1]) + (a2_ref[0] + a2_ref[1])
        x1 = _dot(v_blk, wv_ref[...]) + _dot(agg, wa_ref[...]) + b1_ref[...]
        h = jnp.maximum(x1, 0.0)
        y = _dot(h, w2_ref[...]) + b2_ref[...]
        v_new = v_blk + _ln(y, g_ref[...], beta_ref[...])
        o_ref[...] = v_new
        if with_pq:
            p_ref[...] = _dot(v_new, ws_ref[...])
            q_ref[...] = _dot(v_new, wr_ref[...])

    wargs = [w1v, w1a, sp['b1'].reshape(1, -1), sp['w2'],
             sp['b2'].reshape(1, -1), sp['g'].reshape(1, -1),
             sp['beta'].reshape(1, -1)]
    extra_args = []
    if with_pq:
        extra_args = [nxt['w1'][D:2 * D], nxt['w1'][2 * D:3 * D]]
    in_specs = ([_row_spec(rows, D),
                 pl.BlockSpec((2, rows, D), lambda i: (0, i, 0)),
                 pl.BlockSpec((2, rows, D), lambda i: (0, i, 0))]
                + [_full_spec(a.shape) for a in wargs]
                + [_full_spec((D, D)) for _ in extra_args])
    n_out = 3 if with_pq else 1
    out_sds = jax.ShapeDtypeStruct((N, D), jnp.float32)
    res = pl.pallas_call(
        body,
        grid=(N // rows,),
        in_specs=in_specs,
        out_specs=[_row_spec(rows, D)] * n_out,
        out_shape=[out_sds] * n_out,
    )(v, agg2, agg2b, *wargs, *extra_args)
    if with_pq:
        return res
    return res[0], None, None


def _decode(v, dec):
    rows = 1000

    def body(v_ref, w1_ref, b1_ref, w2_ref, b2_ref, o_ref):
        o_ref[...] = _mlp_block(v_ref[...], w1_ref[...], b1_ref[...],
                                w2_ref[...], b2_ref[...])

    wargs = [dec['w1'], dec['b1'].reshape(1, -1), dec['w2'],
             dec['b2'].reshape(1, -1)]
    wspecs = [_full_spec(a.shape) for a in wargs]
    out_cols = dec['w2'].shape[1]
    return pl.pallas_call(
        body,
        grid=(N // rows,),
        in_specs=[_row_spec(rows, D)] + wspecs,
        out_specs=_row_spec(rows, out_cols),
        out_shape=jax.ShapeDtypeStruct((N, out_cols), jnp.float32),
    )(v, *wargs)


@functools.cache
def _sc_mesh():
    return plsc.VectorSubcoreMesh(
        core_axis_name="c", subcore_axis_name="s",
        num_cores=NC, num_subcores=NS)


@functools.cache
def _sc_gather_kernel(M):
    nb = M // C
    iters = -(-nb // NW)

    @functools.partial(
        pl.kernel,
        out_type=jax.ShapeDtypeStruct((M, D), jnp.float32),
        mesh=_sc_mesh(),
        scratch_types=[
            pltpu.VMEM((C,), jnp.int32), pltpu.VMEM((C,), jnp.int32),
            pltpu.VMEM((C,), jnp.int32), pltpu.VMEM((C,), jnp.int32),
            pltpu.VMEM((C, D), jnp.float32),
            pltpu.VMEM((C, D), jnp.float32), pltpu.VMEM((C, D), jnp.float32),
            pltpu.VMEM_SHARED((N, D), jnp.float32),
            pltpu.SemaphoreType.DMA, pltpu.SemaphoreType.DMA,
            pltpu.SemaphoreType.DMA, pltpu.SemaphoreType.DMA,
            pltpu.SemaphoreType.DMA, pltpu.SemaphoreType.DMA,
        ],
    )
    def gather(p_hbm, q_hbm, s_hbm, r_hbm, g_hbm,
               sidx0, sidx1, ridx0, ridx1, bufp, bufq0, bufq1,
               ptab_sh, semi0, semi1, semg0, semg1, semw0, semw1):
        cid = lax.axis_index("c")
        sid = lax.axis_index("s")
        wid = sid * NC + cid
        sidx = (sidx0, sidx1)
        ridx = (ridx0, ridx1)
        bufq = (bufq0, bufq1)
        semi = (semi0, semi1)
        semg = (semg0, semg1)
        semw = (semw0, semw1)

        def blk(j):
            return wid + j * NW

        def start_idx(j, p):
            @pl.when(blk(j) < nb)
            def _():
                base = blk(j) * C
                pltpu.async_copy(s_hbm.at[pl.ds(base, C)], sidx[p], semi[p])
                pltpu.async_copy(r_hbm.at[pl.ds(base, C)], ridx[p], semi[p])

        def wait_idx(j, p):
            @pl.when(blk(j) < nb)
            def _():
                pltpu.make_async_copy(
                    s_hbm.at[pl.ds(0, C)], sidx[p], semi[p]).wait()
                pltpu.make_async_copy(
                    r_hbm.at[pl.ds(0, C)], ridx[p], semi[p]).wait()

        def start_gather(j, p):
            @pl.when(blk(j) < nb)
            def _():
                pltpu.async_copy(q_hbm.at[ridx[p]], bufq[p], semg[p])

        def wait_gather(j, p):
            @pl.when(blk(j) < nb)
            def _():
                pltpu.make_async_copy(
                    q_hbm.at[ridx[p]], bufq[p], semg[p]).wait()

        def add_p_rows(j, p):
            @pl.when(blk(j) < nb)
            def _():
                pltpu.async_copy(ptab_sh.at[sidx[p]], bufp, semg[p]).wait()

                def row(r, carry):
                    for k in range(D // 16):
                        sl = (r, pl.ds(k * 16, 16))
                        bufq[p][sl] = bufq[p][sl] + bufp[sl]
                    return carry

                lax.fori_loop(0, C, row, 0)

        def start_write(j, p):
            @pl.when(blk(j) < nb)
            def _():
                base = blk(j) * C
                pltpu.async_copy(bufq[p], g_hbm.at[pl.ds(base, C)], semw[p])

        def wait_write(j, p, extra_cond):
            @pl.when(jnp.logical_and(extra_cond, blk(j) < nb))
            def _():
                pltpu.make_async_copy(
                    bufq[p], g_hbm.at[pl.ds(0, C)], semw[p]).wait()

        @pl.when(sid == 0)
        def _():
            pltpu.sync_copy(p_hbm, ptab_sh)

        plsc.subcore_barrier()

        start_idx(0, 0)
        start_idx(1, 1)
        wait_idx(0, 0)
        start_gather(0, 0)

        def body(g, carry):
            for s in (0, 1):
                i = 2 * g + s
                p = s
                o = 1 - s
                wait_gather(i, p)
                add_p_rows(i, p)
                start_write(i, p)
                wait_idx(i + 1, o)
                wait_write(i - 1, o, i >= 1)
                start_gather(i + 1, o)
                start_idx(i + 2, p)
            return carry

        lax.fori_loop(0, (iters + 1) // 2, body, 0)

    return gather


def _sc_gather(p_tab, q_tab, senders, receivers):
    return _sc_gather_kernel(senders.shape[0])(p_tab, q_tab, senders,
                                               receivers)


_BLK_PER_SC = NB // NC
_SC_ITERS = -(-_BLK_PER_SC // NS)
_WB_ROWS = 80
_WB_BLOCKS = N // _WB_ROWS
_WB_ITERS = -(-_WB_BLOCKS // NS)


@functools.cache
def _sc_scatter_kernel(M):
    blk_per_sc = (M // C) // NC

    @functools.partial(
        pl.kernel,
        out_type=jax.ShapeDtypeStruct((NC, N, D), jnp.float32),
        mesh=_sc_mesh(),
        scratch_types=[
            pltpu.VMEM((C,), jnp.int32), pltpu.VMEM((C,), jnp.int32),
            pltpu.VMEM((C, D), jnp.float32), pltpu.VMEM((C, D), jnp.float32),
            pltpu.VMEM_SHARED((N, D), jnp.float32),
            pltpu.VMEM((_WB_ROWS, D), jnp.float32),
            pltpu.SemaphoreType.DMA, pltpu.SemaphoreType.DMA,
        ],
    )
    def scatter(e_hbm, r_hbm, zeros_hbm, out_hbm, ridx0, ridx1,
                buf0, buf1, acc, obuf, seml0, seml1):
        cid = lax.axis_index("c")
        sid = lax.axis_index("s")
        ridx = (ridx0, ridx1)
        buf = (buf0, buf1)
        seml = (seml0, seml1)

        @pl.when(sid == 0)
        def _():
            pltpu.sync_copy(zeros_hbm, acc)

        plsc.subcore_barrier()

        def base_of(j):
            return cid * (M // NC) + (sid + j * NS) * C

        def in_range(j):
            return (sid + j * NS) < blk_per_sc

        def start_load(j, p):
            @pl.when(in_range(j))
            def _():
                base = base_of(j)
                pltpu.async_copy(r_hbm.at[pl.ds(base, C)], ridx[p], seml[p])
                pltpu.async_copy(e_hbm.at[pl.ds(base, C)], buf[p], seml[p])

        def wait_load(j, p):
            @pl.when(in_range(j))
            def _():
                pltpu.make_async_copy(
                    r_hbm.at[pl.ds(0, C)], ridx[p], seml[p]).wait()
                pltpu.make_async_copy(
                    e_hbm.at[pl.ds(0, C)], buf[p], seml[p]).wait()

        def do_add(j, p):
            @pl.when(in_range(j))
            def _():
                pltpu.sync_copy(buf[p], acc.at[ridx[p]], add=True)

        start_load(0, 0)
        start_load(1, 1)

        def body(g, carry):
            for s in (0, 1):
                i = 2 * g + s
                wait_load(i, s)
                do_add(i, s)
                start_load(i + 2, s)
            return carry

        lax.fori_loop(0, (-(-blk_per_sc // NS) + 1) // 2, body, 0)
        plsc.subcore_barrier()

        def wb_body(i, carry):
            b = sid + i * NS

            @pl.when(b < _WB_BLOCKS)
            def _():
                row0 = b * _WB_ROWS
                pltpu.sync_copy(acc.at[pl.ds(row0, _WB_ROWS)], obuf)
                pltpu.sync_copy(obuf, out_hbm.at[cid, pl.ds(row0, _WB_ROWS)])

            return carry

        lax.fori_loop(0, _WB_ITERS, wb_body, 0)

    return scatter


def _sc_scatter(e, receivers, zeros):
    return _sc_scatter_kernel(receivers.shape[0])(e, receivers, zeros)


def kernel(node_features, edge_features, senders, receivers, params):
    steps = params['steps']
    v, p_tab, q_tab = _node_encode(
        node_features, params['node_enc'],
        steps[0]['edge']['w1'][D:2 * D], steps[0]['edge']['w1'][2 * D:3 * D])
    zeros = jnp.zeros((N, D), jnp.float32)
    h = E // 2
    s0, s1 = senders[:h], senders[h:]
    r0, r1 = receivers[:h], receivers[h:]
    ef0, ef1 = edge_features[:h], edge_features[h:]
    e0 = e1 = None
    for i, sp in enumerate(steps):
        g0 = _sc_gather(p_tab, q_tab, s0, r0)
        g1 = _sc_gather(p_tab, q_tab, s1, r1)
        if i == 0:
            e0 = _edge_step(None, g0, sp['edge'],
                            enc=params['edge_enc'], raw=ef0)
            e1 = _edge_step(None, g1, sp['edge'],
                            enc=params['edge_enc'], raw=ef1)
        else:
            e0 = _edge_step(e0, g0, sp['edge'])
            e1 = _edge_step(e1, g1, sp['edge'])
        a0 = _sc_scatter(e0, r0, zeros)
        a1 = _sc_scatter(e1, r1, zeros)
        nxt = steps[i + 1]['edge'] if i + 1 < len(steps) else None
        v, p_tab, q_tab = _node_step(v, a0, a1, sp['node'], nxt)
    return _decode(v, params['decoder'])


# speedup vs baseline: 5.4358x; 1.0565x over previous
import functools

import jax
import jax.numpy as jnp
from jax import lax
from jax.experimental import pallas as pl
from jax.experimental.pallas import tpu as pltpu
from jax.experimental.pallas import tpu_sc as plsc

N = 10000
E = 320000
D = 128

NC = 2
NS = 16
NW = NC * NS

C = 128
NB = E // C
ITERS = -(-NB // NW)

PREC = jax.lax.Precision.DEFAULT


def _ln(y, g, beta):
    mu = jnp.mean(y, axis=-1, keepdims=True)
    var = jnp.mean((y - mu) * (y - mu), axis=-1, keepdims=True)
    return (y - mu) * lax.rsqrt(var + 1e-5) * g + beta


def _dot(a, b):
    return jnp.dot(a, b, preferred_element_type=jnp.float32, precision=PREC)


def _row_spec(rows, cols):
    return pl.BlockSpec((rows, cols), lambda i: (i, 0))


def _full_spec(shape):
    nd = len(shape)
    return pl.BlockSpec(shape, lambda i: (0,) * nd)


def _weight_args(p, din):
    args = [p['w1'], p['b1'].reshape(1, -1), p['w2'], p['b2'].reshape(1, -1)]
    if 'g' in p:
        args += [p['g'].reshape(1, -1), p['beta'].reshape(1, -1)]
    specs = [_full_spec(a.shape) for a in args]
    return args, specs


def _mlp_block(x, w1, b1, w2, b2, g=None, beta=None, extra=None):
    x1 = _dot(x, w1) + b1
    if extra is not None:
        x1 = x1 + extra
    h = jnp.maximum(x1, 0.0)
    y = _dot(h, w2) + b2
    if g is not None:
        y = _ln(y, g, beta)
    return y


def _node_encode(x, enc, w1s, w1r):
    rows = 1000

    def body(x_ref, w1_ref, b1_ref, w2_ref, b2_ref, g_ref, beta_ref,
             ws_ref, wr_ref, v_ref, p_ref, q_ref):
        v = _mlp_block(x_ref[...], w1_ref[...], b1_ref[...], w2_ref[...],
                       b2_ref[...], g_ref[...], beta_ref[...])
        v_ref[...] = v
        p_ref[...] = _dot(v, ws_ref[...])
        q_ref[...] = _dot(v, wr_ref[...])

    wargs, wspecs = _weight_args(enc, x.shape[1])
    out_sds = jax.ShapeDtypeStruct((N, D), jnp.float32)
    return pl.pallas_call(
        body,
        grid=(N // rows,),
        in_specs=[_row_spec(rows, x.shape[1])] + wspecs
        + [_full_spec((D, D)), _full_spec((D, D))],
        out_specs=[_row_spec(rows, D)] * 3,
        out_shape=[out_sds] * 3,
    )(x, *wargs, w1s, w1r)


def _edge_encode(x, enc):
    rows = 2000

    def body(x_ref, w1_ref, b1_ref, w2_ref, b2_ref, g_ref, beta_ref, o_ref):
        o_ref[...] = _mlp_block(x_ref[...], w1_ref[...], b1_ref[...],
                                w2_ref[...], b2_ref[...], g_ref[...],
                                beta_ref[...])

    wargs, wspecs = _weight_args(enc, x.shape[1])
    return pl.pallas_call(
        body,
        grid=(E // rows,),
        in_specs=[_row_spec(rows, x.shape[1])] + wspecs,
        out_specs=_row_spec(rows, D),
        out_shape=jax.ShapeDtypeStruct((E, D), jnp.float32),
    )(x, *wargs)


def _edge_step(e, gsum, sp, enc=None, raw=None):
    rows = 2000
    w1e = sp['w1'][0:D]
    fused_enc = enc is not None

    def body(*refs):
        if fused_enc:
            (x_ref, gs_ref, ew1, eb1, ew2, eb2, eg, ebeta,
             w1_ref, b1_ref, w2_ref, b2_ref, g_ref, beta_ref, o_ref) = refs
            e_blk = _mlp_block(x_ref[...], ew1[...], eb1[...], ew2[...],
                               eb2[...], eg[...], ebeta[...])
        else:
            (e_ref, gs_ref, w1_ref, b1_ref, w2_ref, b2_ref,
             g_ref, beta_ref, o_ref) = refs
            e_blk = e_ref[...]
        y = _mlp_block(e_blk, w1_ref[...], b1_ref[...], w2_ref[...],
                       b2_ref[...], g_ref[...], beta_ref[...],
                       extra=gs_ref[...])
        o_ref[...] = e_blk + y

    wargs = [w1e, sp['b1'].reshape(1, -1), sp['w2'], sp['b2'].reshape(1, -1),
             sp['g'].reshape(1, -1), sp['beta'].reshape(1, -1)]
    if fused_enc:
        eargs, especs = _weight_args(enc, raw.shape[1])
        first = raw
        in_specs = ([_row_spec(rows, raw.shape[1]), _row_spec(rows, D)]
                    + especs + [_full_spec(a.shape) for a in wargs])
        args = [raw, gsum] + eargs + wargs
    else:
        in_specs = ([_row_spec(rows, D)] * 2
                    + [_full_spec(a.shape) for a in wargs])
        args = [e, gsum] + wargs
    m = args[0].shape[0]
    return pl.pallas_call(
        body,
        grid=(m // rows,),
        in_specs=in_specs,
        out_specs=_row_spec(rows, D),
        out_shape=jax.ShapeDtypeStruct((m, D), jnp.float32),
    )(*args)


def _node_step(v, agg2, agg2b, sp, nxt):
    rows = 1000
    w1v = sp['w1'][0:D]
    w1a = sp['w1'][D:2 * D]
    with_pq = nxt is not None

    def body(v_ref, a_ref, a2_ref, wv_ref, wa_ref, b1_ref, w2_ref, b2_ref,
             g_ref, beta_ref, *rest):
        if with_pq:
            ws_ref, wr_ref, o_ref, p_ref, q_ref = rest
        else:
            (o_ref,) = rest
        v_blk = v_ref[...]
        agg = (a_ref[0] + a_ref[1]) + (a2_ref[0] + a2_ref[1])
        x1 = _dot(v_blk, wv_ref[...]) + _dot(agg, wa_ref[...]) + b1_ref[...]
        h = jnp.maximum(x1, 0.0)
        y = _dot(h, w2_ref[...]) + b2_ref[...]
        v_new = v_blk + _ln(y, g_ref[...], beta_ref[...])
        o_ref[...] = v_new
        if with_pq:
            p_ref[...] = _dot(v_new, ws_ref[...])
            q_ref[...] = _dot(v_new, wr_ref[...])

    wargs = [w1v, w1a, sp['b1'].reshape(1, -1), sp['w2'],
             sp['b2'].reshape(1, -1), sp['g'].reshape(1, -1),
             sp['beta'].reshape(1, -1)]
    extra_args = []
    if with_pq:
        extra_args = [nxt['w1'][D:2 * D], nxt['w1'][2 * D:3 * D]]
    in_specs = ([_row_spec(rows, D),
                 pl.BlockSpec((2, rows, D), lambda i: (0, i, 0)),
                 pl.BlockSpec((2, rows, D), lambda i: (0, i, 0))]
                + [_full_spec(a.shape) for a in wargs]
                + [_full_spec((D, D)) for _ in extra_args])
    n_out = 3 if with_pq else 1
    out_sds = jax.ShapeDtypeStruct((N, D), jnp.float32)
    res = pl.pallas_call(
        body,
        grid=(N // rows,),
        in_specs=in_specs,
        out_specs=[_row_spec(rows, D)] * n_out,
        out_shape=[out_sds] * n_out,
    )(v, agg2, agg2b, *wargs, *extra_args)
    if with_pq:
        return res
    return res[0], None, None


def _decode(v, dec):
    rows = 1000

    def body(v_ref, w1_ref, b1_ref, w2_ref, b2_ref, o_ref):
        o_ref[...] = _mlp_block(v_ref[...], w1_ref[...], b1_ref[...],
                                w2_ref[...], b2_ref[...])

    wargs = [dec['w1'], dec['b1'].reshape(1, -1), dec['w2'],
             dec['b2'].reshape(1, -1)]
    wspecs = [_full_spec(a.shape) for a in wargs]
    out_cols = dec['w2'].shape[1]
    return pl.pallas_call(
        body,
        grid=(N // rows,),
        in_specs=[_row_spec(rows, D)] + wspecs,
        out_specs=_row_spec(rows, out_cols),
        out_shape=jax.ShapeDtypeStruct((N, out_cols), jnp.float32),
    )(v, *wargs)


@functools.cache
def _sc_mesh():
    return plsc.VectorSubcoreMesh(
        core_axis_name="c", subcore_axis_name="s",
        num_cores=NC, num_subcores=NS)


@functools.cache
def _sc_gather_kernel(M):
    nb = M // C
    iters = -(-nb // NW)

    @functools.partial(
        pl.kernel,
        out_type=jax.ShapeDtypeStruct((M, D), jnp.float32),
        mesh=_sc_mesh(),
        scratch_types=[
            pltpu.VMEM((C,), jnp.int32), pltpu.VMEM((C,), jnp.int32),
            pltpu.VMEM((C,), jnp.int32), pltpu.VMEM((C,), jnp.int32),
            pltpu.VMEM((C, D), jnp.float32),
            pltpu.VMEM((C, D), jnp.float32), pltpu.VMEM((C, D), jnp.float32),
            pltpu.VMEM_SHARED((N, D), jnp.float32),
            pltpu.SemaphoreType.DMA, pltpu.SemaphoreType.DMA,
            pltpu.SemaphoreType.DMA, pltpu.SemaphoreType.DMA,
            pltpu.SemaphoreType.DMA, pltpu.SemaphoreType.DMA,
            pltpu.SemaphoreType.DMA,
        ],
    )
    def gather(p_hbm, q_hbm, s_hbm, r_hbm, g_hbm,
               sidx0, sidx1, ridx0, ridx1, bufp, bufq0, bufq1,
               ptab_sh, semi0, semi1, semg0, semg1, semw0, semw1, semp):
        cid = lax.axis_index("c")
        sid = lax.axis_index("s")
        wid = sid * NC + cid
        sidx = (sidx0, sidx1)
        ridx = (ridx0, ridx1)
        bufq = (bufq0, bufq1)
        semi = (semi0, semi1)
        semg = (semg0, semg1)
        semw = (semw0, semw1)

        def blk(j):
            return wid + j * NW

        def start_idx(j, p):
            @pl.when(blk(j) < nb)
            def _():
                base = blk(j) * C
                pltpu.async_copy(s_hbm.at[pl.ds(base, C)], sidx[p], semi[p])
                pltpu.async_copy(r_hbm.at[pl.ds(base, C)], ridx[p], semi[p])

        def wait_idx(j, p):
            @pl.when(blk(j) < nb)
            def _():
                pltpu.make_async_copy(
                    s_hbm.at[pl.ds(0, C)], sidx[p], semi[p]).wait()
                pltpu.make_async_copy(
                    r_hbm.at[pl.ds(0, C)], ridx[p], semi[p]).wait()

        def start_gather(j, p):
            @pl.when(blk(j) < nb)
            def _():
                pltpu.async_copy(q_hbm.at[ridx[p]], bufq[p], semg[p])

        def wait_gather(j, p):
            @pl.when(blk(j) < nb)
            def _():
                pltpu.make_async_copy(
                    q_hbm.at[ridx[p]], bufq[p], semg[p]).wait()

        def start_pg(j, p):
            @pl.when(blk(j) < nb)
            def _():
                pltpu.async_copy(ptab_sh.at[sidx[p]], bufp, semp)

        def add_p_rows(j, p):
            @pl.when(blk(j) < nb)
            def _():
                pltpu.make_async_copy(ptab_sh.at[sidx[p]], bufp, semp).wait()

                def row(r, carry):
                    for k in range(D // 16):
                        sl = (r, pl.ds(k * 16, 16))
                        bufq[p][sl] = bufq[p][sl] + bufp[sl]
                    return carry

                lax.fori_loop(0, C, row, 0)

        def start_write(j, p):
            @pl.when(blk(j) < nb)
            def _():
                base = blk(j) * C
                pltpu.async_copy(bufq[p], g_hbm.at[pl.ds(base, C)], semw[p])

        def wait_write(j, p, extra_cond):
            @pl.when(jnp.logical_and(extra_cond, blk(j) < nb))
            def _():
                pltpu.make_async_copy(
                    bufq[p], g_hbm.at[pl.ds(0, C)], semw[p]).wait()

        @pl.when(sid == 0)
        def _():
            pltpu.sync_copy(p_hbm, ptab_sh)

        plsc.subcore_barrier()

        start_idx(0, 0)
        start_idx(1, 1)
        wait_idx(0, 0)
        start_gather(0, 0)
        start_pg(0, 0)

        def body(g, carry):
            for s in (0, 1):
                i = 2 * g + s
                p = s
                o = 1 - s
                wait_gather(i, p)
                add_p_rows(i, p)
                start_write(i, p)
                wait_idx(i + 1, o)
                wait_write(i - 1, o, i >= 1)
                start_gather(i + 1, o)
                start_pg(i + 1, o)
                start_idx(i + 2, p)
            return carry

        lax.fori_loop(0, (iters + 1) // 2, body, 0)

    return gather


def _sc_gather(p_tab, q_tab, senders, receivers):
    return _sc_gather_kernel(senders.shape[0])(p_tab, q_tab, senders,
                                               receivers)


_BLK_PER_SC = NB // NC
_SC_ITERS = -(-_BLK_PER_SC // NS)
_WB_ROWS = 80
_WB_BLOCKS = N // _WB_ROWS
_WB_ITERS = -(-_WB_BLOCKS // NS)


@functools.cache
def _sc_scatter_kernel(M):
    blk_per_sc = (M // C) // NC

    @functools.partial(
        pl.kernel,
        out_type=jax.ShapeDtypeStruct((NC, N, D), jnp.float32),
        mesh=_sc_mesh(),
        scratch_types=[
            pltpu.VMEM((C,), jnp.int32), pltpu.VMEM((C,), jnp.int32),
            pltpu.VMEM((C, D), jnp.float32), pltpu.VMEM((C, D), jnp.float32),
            pltpu.VMEM_SHARED((N, D), jnp.float32),
            pltpu.VMEM((_WB_ROWS, D), jnp.float32),
            pltpu.SemaphoreType.DMA, pltpu.SemaphoreType.DMA,
        ],
    )
    def scatter(e_hbm, r_hbm, zeros_hbm, out_hbm, ridx0, ridx1,
                buf0, buf1, acc, obuf, seml0, seml1):
        cid = lax.axis_index("c")
        sid = lax.axis_index("s")
        ridx = (ridx0, ridx1)
        buf = (buf0, buf1)
        seml = (seml0, seml1)

        @pl.when(sid == 0)
        def _():
            pltpu.sync_copy(zeros_hbm, acc)

        plsc.subcore_barrier()

        def base_of(j):
            return cid * (M // NC) + (sid + j * NS) * C

        def in_range(j):
            return (sid + j * NS) < blk_per_sc

        def start_load(j, p):
            @pl.when(in_range(j))
            def _():
                base = base_of(j)
                pltpu.async_copy(r_hbm.at[pl.ds(base, C)], ridx[p], seml[p])
                pltpu.async_copy(e_hbm.at[pl.ds(base, C)], buf[p], seml[p])

        def wait_load(j, p):
            @pl.when(in_range(j))
            def _():
                pltpu.make_async_copy(
                    r_hbm.at[pl.ds(0, C)], ridx[p], seml[p]).wait()
                pltpu.make_async_copy(
                    e_hbm.at[pl.ds(0, C)], buf[p], seml[p]).wait()

        def do_add(j, p):
            @pl.when(in_range(j))
            def _():
                pltpu.sync_copy(buf[p], acc.at[ridx[p]], add=True)

        start_load(0, 0)
        start_load(1, 1)

        def body(g, carry):
            for s in (0, 1):
                i = 2 * g + s
                wait_load(i, s)
                do_add(i, s)
                start_load(i + 2, s)
            return carry

        lax.fori_loop(0, (-(-blk_per_sc // NS) + 1) // 2, body, 0)
        plsc.subcore_barrier()

        def wb_body(i, carry):
            b = sid + i * NS

            @pl.when(b < _WB_BLOCKS)
            def _():
                row0 = b * _WB_ROWS
                pltpu.sync_copy(acc.at[pl.ds(row0, _WB_ROWS)], obuf)
                pltpu.sync_copy(obuf, out_hbm.at[cid, pl.ds(row0, _WB_ROWS)])

            return carry

        lax.fori_loop(0, _WB_ITERS, wb_body, 0)

    return scatter


def _sc_scatter(e, receivers, zeros):
    return _sc_scatter_kernel(receivers.shape[0])(e, receivers, zeros)


def kernel(node_features, edge_features, senders, receivers, params):
    steps = params['steps']
    v, p_tab, q_tab = _node_encode(
        node_features, params['node_enc'],
        steps[0]['edge']['w1'][D:2 * D], steps[0]['edge']['w1'][2 * D:3 * D])
    zeros = jnp.zeros((N, D), jnp.float32)
    h = E // 2
    s0, s1 = senders[:h], senders[h:]
    r0, r1 = receivers[:h], receivers[h:]
    ef0, ef1 = edge_features[:h], edge_features[h:]
    e0 = e1 = None
    for i, sp in enumerate(steps):
        g0 = _sc_gather(p_tab, q_tab, s0, r0)
        g1 = _sc_gather(p_tab, q_tab, s1, r1)
        if i == 0:
            e0 = _edge_step(None, g0, sp['edge'],
                            enc=params['edge_enc'], raw=ef0)
            e1 = _edge_step(None, g1, sp['edge'],
                            enc=params['edge_enc'], raw=ef1)
        else:
            e0 = _edge_step(e0, g0, sp['edge'])
            e1 = _edge_step(e1, g1, sp['edge'])
        a0 = _sc_scatter(e0, r0, zeros)
        a1 = _sc_scatter(e1, r1, zeros)
        nxt = steps[i + 1]['edge'] if i + 1 < len(steps) else None
        v, p_tab, q_tab = _node_step(v, a0, a1, sp['node'], nxt)
    return _decode(v, params['decoder'])


# speedup vs baseline: 5.4450x; 1.0017x over previous
import functools

import jax
import jax.numpy as jnp
from jax import lax
from jax.experimental import pallas as pl
from jax.experimental.pallas import tpu as pltpu
from jax.experimental.pallas import tpu_sc as plsc

N = 10000
E = 320000
D = 128

NC = 2
NS = 16
NW = NC * NS

C = 128
NB = E // C
ITERS = -(-NB // NW)

PREC = jax.lax.Precision.DEFAULT


def _ln(y, g, beta):
    mu = jnp.mean(y, axis=-1, keepdims=True)
    var = jnp.mean((y - mu) * (y - mu), axis=-1, keepdims=True)
    return (y - mu) * lax.rsqrt(var + 1e-5) * g + beta


def _dot(a, b):
    return jnp.dot(a, b, preferred_element_type=jnp.float32, precision=PREC)


def _row_spec(rows, cols):
    return pl.BlockSpec((rows, cols), lambda i: (i, 0))


def _full_spec(shape):
    nd = len(shape)
    return pl.BlockSpec(shape, lambda i: (0,) * nd)


def _weight_args(p, din):
    args = [p['w1'], p['b1'].reshape(1, -1), p['w2'], p['b2'].reshape(1, -1)]
    if 'g' in p:
        args += [p['g'].reshape(1, -1), p['beta'].reshape(1, -1)]
    specs = [_full_spec(a.shape) for a in args]
    return args, specs


def _mlp_block(x, w1, b1, w2, b2, g=None, beta=None, extra=None):
    x1 = _dot(x, w1) + b1
    if extra is not None:
        x1 = x1 + extra
    h = jnp.maximum(x1, 0.0)
    y = _dot(h, w2) + b2
    if g is not None:
        y = _ln(y, g, beta)
    return y


def _node_encode(x, enc, w1s, w1r):
    rows = 1000

    def body(x_ref, w1_ref, b1_ref, w2_ref, b2_ref, g_ref, beta_ref,
             ws_ref, wr_ref, v_ref, p_ref, q_ref):
        v = _mlp_block(x_ref[...], w1_ref[...], b1_ref[...], w2_ref[...],
                       b2_ref[...], g_ref[...], beta_ref[...])
        v_ref[...] = v
        p_ref[...] = _dot(v, ws_ref[...])
        q_ref[...] = _dot(v, wr_ref[...])

    wargs, wspecs = _weight_args(enc, x.shape[1])
    out_sds = jax.ShapeDtypeStruct((N, D), jnp.float32)
    return pl.pallas_call(
        body,
        grid=(N // rows,),
        in_specs=[_row_spec(rows, x.shape[1])] + wspecs
        + [_full_spec((D, D)), _full_spec((D, D))],
        out_specs=[_row_spec(rows, D)] * 3,
        out_shape=[out_sds] * 3,
    )(x, *wargs, w1s, w1r)


def _edge_encode(x, enc):
    rows = 2000

    def body(x_ref, w1_ref, b1_ref, w2_ref, b2_ref, g_ref, beta_ref, o_ref):
        o_ref[...] = _mlp_block(x_ref[...], w1_ref[...], b1_ref[...],
                                w2_ref[...], b2_ref[...], g_ref[...],
                                beta_ref[...])

    wargs, wspecs = _weight_args(enc, x.shape[1])
    return pl.pallas_call(
        body,
        grid=(E // rows,),
        in_specs=[_row_spec(rows, x.shape[1])] + wspecs,
        out_specs=_row_spec(rows, D),
        out_shape=jax.ShapeDtypeStruct((E, D), jnp.float32),
    )(x, *wargs)


def _edge_step(e, gsum, sp, enc=None, raw=None):
    rows = 2000
    w1e = sp['w1'][0:D]
    fused_enc = enc is not None

    def body(*refs):
        if fused_enc:
            (x_ref, gs_ref, ew1, eb1, ew2, eb2, eg, ebeta,
             w1_ref, b1_ref, w2_ref, b2_ref, g_ref, beta_ref, o_ref) = refs
            e_blk = _mlp_block(x_ref[...], ew1[...], eb1[...], ew2[...],
                               eb2[...], eg[...], ebeta[...])
        else:
            (e_ref, gs_ref, w1_ref, b1_ref, w2_ref, b2_ref,
             g_ref, beta_ref, o_ref) = refs
            e_blk = e_ref[...]
        y = _mlp_block(e_blk, w1_ref[...], b1_ref[...], w2_ref[...],
                       b2_ref[...], g_ref[...], beta_ref[...],
                       extra=gs_ref[...])
        o_ref[...] = e_blk + y

    wargs = [w1e, sp['b1'].reshape(1, -1), sp['w2'], sp['b2'].reshape(1, -1),
             sp['g'].reshape(1, -1), sp['beta'].reshape(1, -1)]
    if fused_enc:
        eargs, especs = _weight_args(enc, raw.shape[1])
        first = raw
        in_specs = ([_row_spec(rows, raw.shape[1]), _row_spec(rows, D)]
                    + especs + [_full_spec(a.shape) for a in wargs])
        args = [raw, gsum] + eargs + wargs
    else:
        in_specs = ([_row_spec(rows, D)] * 2
                    + [_full_spec(a.shape) for a in wargs])
        args = [e, gsum] + wargs
    m = args[0].shape[0]
    return pl.pallas_call(
        body,
        grid=(m // rows,),
        in_specs=in_specs,
        out_specs=_row_spec(rows, D),
        out_shape=jax.ShapeDtypeStruct((m, D), jnp.float32),
    )(*args)


def _node_step(v, aggs, sp, nxt):
    rows = 1000
    w1v = sp['w1'][0:D]
    w1a = sp['w1'][D:2 * D]
    with_pq = nxt is not None

    n_agg = len(aggs)

    def body(v_ref, *rest):
        a_refs = rest[:n_agg]
        (wv_ref, wa_ref, b1_ref, w2_ref, b2_ref, g_ref, beta_ref,
         *rest2) = rest[n_agg:]
        if with_pq:
            ws_ref, wr_ref, o_ref, p_ref, q_ref = rest2
        else:
            (o_ref,) = rest2
        v_blk = v_ref[...]
        agg = a_refs[0][0] + a_refs[0][1]
        for a in a_refs[1:]:
            agg = agg + a[0] + a[1]
        x1 = _dot(v_blk, wv_ref[...]) + _dot(agg, wa_ref[...]) + b1_ref[...]
        h = jnp.maximum(x1, 0.0)
        y = _dot(h, w2_ref[...]) + b2_ref[...]
        v_new = v_blk + _ln(y, g_ref[...], beta_ref[...])
        o_ref[...] = v_new
        if with_pq:
            p_ref[...] = _dot(v_new, ws_ref[...])
            q_ref[...] = _dot(v_new, wr_ref[...])

    wargs = [w1v, w1a, sp['b1'].reshape(1, -1), sp['w2'],
             sp['b2'].reshape(1, -1), sp['g'].reshape(1, -1),
             sp['beta'].reshape(1, -1)]
    extra_args = []
    if with_pq:
        extra_args = [nxt['w1'][D:2 * D], nxt['w1'][2 * D:3 * D]]
    in_specs = ([_row_spec(rows, D)]
                + [pl.BlockSpec((2, rows, D), lambda i: (0, i, 0))
                   for _ in aggs]
                + [_full_spec(a.shape) for a in wargs]
                + [_full_spec((D, D)) for _ in extra_args])
    n_out = 3 if with_pq else 1
    out_sds = jax.ShapeDtypeStruct((N, D), jnp.float32)
    res = pl.pallas_call(
        body,
        grid=(N // rows,),
        in_specs=in_specs,
        out_specs=[_row_spec(rows, D)] * n_out,
        out_shape=[out_sds] * n_out,
    )(v, *aggs, *wargs, *extra_args)
    if with_pq:
        return res
    return res[0], None, None


def _decode(v, dec):
    rows = 1000

    def body(v_ref, w1_ref, b1_ref, w2_ref, b2_ref, o_ref):
        o_ref[...] = _mlp_block(v_ref[...], w1_ref[...], b1_ref[...],
                                w2_ref[...], b2_ref[...])

    wargs = [dec['w1'], dec['b1'].reshape(1, -1), dec['w2'],
             dec['b2'].reshape(1, -1)]
    wspecs = [_full_spec(a.shape) for a in wargs]
    out_cols = dec['w2'].shape[1]
    return pl.pallas_call(
        body,
        grid=(N // rows,),
        in_specs=[_row_spec(rows, D)] + wspecs,
        out_specs=_row_spec(rows, out_cols),
        out_shape=jax.ShapeDtypeStruct((N, out_cols), jnp.float32),
    )(v, *wargs)


@functools.cache
def _sc_mesh():
    return plsc.VectorSubcoreMesh(
        core_axis_name="c", subcore_axis_name="s",
        num_cores=NC, num_subcores=NS)


@functools.cache
def _sc_gather_kernel(M):
    nb = M // C
    iters = -(-nb // NW)

    @functools.partial(
        pl.kernel,
        out_type=jax.ShapeDtypeStruct((M, D), jnp.float32),
        mesh=_sc_mesh(),
        scratch_types=[
            pltpu.VMEM((C,), jnp.int32), pltpu.VMEM((C,), jnp.int32),
            pltpu.VMEM((C,), jnp.int32), pltpu.VMEM((C,), jnp.int32),
            pltpu.VMEM((C, D), jnp.float32),
            pltpu.VMEM((C, D), jnp.float32), pltpu.VMEM((C, D), jnp.float32),
            pltpu.VMEM_SHARED((N, D), jnp.float32),
            pltpu.SemaphoreType.DMA, pltpu.SemaphoreType.DMA,
            pltpu.SemaphoreType.DMA, pltpu.SemaphoreType.DMA,
            pltpu.SemaphoreType.DMA, pltpu.SemaphoreType.DMA,
            pltpu.SemaphoreType.DMA,
        ],
    )
    def gather(p_hbm, q_hbm, s_hbm, r_hbm, g_hbm,
               sidx0, sidx1, ridx0, ridx1, bufp, bufq0, bufq1,
               ptab_sh, semi0, semi1, semg0, semg1, semw0, semw1, semp):
        cid = lax.axis_index("c")
        sid = lax.axis_index("s")
        wid = sid * NC + cid
        sidx = (sidx0, sidx1)
        ridx = (ridx0, ridx1)
        bufq = (bufq0, bufq1)
        semi = (semi0, semi1)
        semg = (semg0, semg1)
        semw = (semw0, semw1)

        def blk(j):
            return wid + j * NW

        def start_idx(j, p):
            @pl.when(blk(j) < nb)
            def _():
                base = blk(j) * C
                pltpu.async_copy(s_hbm.at[pl.ds(base, C)], sidx[p], semi[p])
                pltpu.async_copy(r_hbm.at[pl.ds(base, C)], ridx[p], semi[p])

        def wait_idx(j, p):
            @pl.when(blk(j) < nb)
            def _():
                pltpu.make_async_copy(
                    s_hbm.at[pl.ds(0, C)], sidx[p], semi[p]).wait()
                pltpu.make_async_copy(
                    r_hbm.at[pl.ds(0, C)], ridx[p], semi[p]).wait()

        def start_gather(j, p):
            @pl.when(blk(j) < nb)
            def _():
                pltpu.async_copy(q_hbm.at[ridx[p]], bufq[p], semg[p])

        def wait_gather(j, p):
            @pl.when(blk(j) < nb)
            def _():
                pltpu.make_async_copy(
                    q_hbm.at[ridx[p]], bufq[p], semg[p]).wait()

        def start_pg(j, p):
            @pl.when(blk(j) < nb)
            def _():
                pltpu.async_copy(ptab_sh.at[sidx[p]], bufp, semp)

        def add_p_rows(j, p):
            @pl.when(blk(j) < nb)
            def _():
                pltpu.make_async_copy(ptab_sh.at[sidx[p]], bufp, semp).wait()

                def row(r, carry):
                    for k in range(D // 16):
                        sl = (r, pl.ds(k * 16, 16))
                        bufq[p][sl] = bufq[p][sl] + bufp[sl]
                    return carry

                lax.fori_loop(0, C, row, 0)

        def start_write(j, p):
            @pl.when(blk(j) < nb)
            def _():
                base = blk(j) * C
                pltpu.async_copy(bufq[p], g_hbm.at[pl.ds(base, C)], semw[p])

        def wait_write(j, p, extra_cond):
            @pl.when(jnp.logical_and(extra_cond, blk(j) < nb))
            def _():
                pltpu.make_async_copy(
                    bufq[p], g_hbm.at[pl.ds(0, C)], semw[p]).wait()

        @pl.when(sid == 0)
        def _():
            pltpu.sync_copy(p_hbm, ptab_sh)

        plsc.subcore_barrier()

        start_idx(0, 0)
        start_idx(1, 1)
        wait_idx(0, 0)
        start_gather(0, 0)
        start_pg(0, 0)

        def body(g, carry):
            for s in (0, 1):
                i = 2 * g + s
                p = s
                o = 1 - s
                wait_gather(i, p)
                add_p_rows(i, p)
                start_write(i, p)
                wait_idx(i + 1, o)
                wait_write(i - 1, o, i >= 1)
                start_gather(i + 1, o)
                start_pg(i + 1, o)
                start_idx(i + 2, p)
            return carry

        lax.fori_loop(0, (iters + 1) // 2, body, 0)

    return gather


def _sc_gather(p_tab, q_tab, senders, receivers):
    return _sc_gather_kernel(senders.shape[0])(p_tab, q_tab, senders,
                                               receivers)


_BLK_PER_SC = NB // NC
_SC_ITERS = -(-_BLK_PER_SC // NS)
_WB_ROWS = 80
_WB_BLOCKS = N // _WB_ROWS
_WB_ITERS = -(-_WB_BLOCKS // NS)


@functools.cache
def _sc_scatter_kernel(M):
    @functools.partial(
        pl.kernel,
        out_type=jax.ShapeDtypeStruct((NC, N, D), jnp.float32),
        mesh=_sc_mesh(),
        scratch_types=[
            pltpu.VMEM((C,), jnp.int32), pltpu.VMEM((C,), jnp.int32),
            pltpu.VMEM((C, D), jnp.float32), pltpu.VMEM((C, D), jnp.float32),
            pltpu.VMEM_SHARED((N, D), jnp.float32),
            pltpu.VMEM((_WB_ROWS, D), jnp.float32),
            pltpu.SemaphoreType.DMA, pltpu.SemaphoreType.DMA,
        ],
    )
    def scatter(e_hbm, r_hbm, zeros_hbm, out_hbm, ridx0, ridx1,
                buf0, buf1, acc, obuf, seml0, seml1):
        cid = lax.axis_index("c")
        sid = lax.axis_index("s")
        wid = sid * NC + cid
        ridx = (ridx0, ridx1)
        buf = (buf0, buf1)
        seml = (seml0, seml1)

        @pl.when(sid == 0)
        def _():
            pltpu.sync_copy(zeros_hbm, acc)

        plsc.subcore_barrier()

        def base_of(j):
            return (wid + j * NW) * C

        def in_range(j):
            return (wid + j * NW) < (M // C)

        def start_load(j, p):
            @pl.when(in_range(j))
            def _():
                base = base_of(j)
                pltpu.async_copy(r_hbm.at[pl.ds(base, C)], ridx[p], seml[p])
                pltpu.async_copy(e_hbm.at[pl.ds(base, C)], buf[p], seml[p])

        def wait_load(j, p):
            @pl.when(in_range(j))
            def _():
                pltpu.make_async_copy(
                    r_hbm.at[pl.ds(0, C)], ridx[p], seml[p]).wait()
                pltpu.make_async_copy(
                    e_hbm.at[pl.ds(0, C)], buf[p], seml[p]).wait()

        def do_add(j, p):
            @pl.when(in_range(j))
            def _():
                pltpu.sync_copy(buf[p], acc.at[ridx[p]], add=True)

        start_load(0, 0)
        start_load(1, 1)

        def body(g, carry):
            for s in (0, 1):
                i = 2 * g + s
                wait_load(i, s)
                do_add(i, s)
                start_load(i + 2, s)
            return carry

        lax.fori_loop(0, (-(-(M // C) // NW) + 1) // 2, body, 0)
        plsc.subcore_barrier()

        def wb_body(i, carry):
            b = sid + i * NS

            @pl.when(b < _WB_BLOCKS)
            def _():
                row0 = b * _WB_ROWS
                pltpu.sync_copy(acc.at[pl.ds(row0, _WB_ROWS)], obuf)
                pltpu.sync_copy(obuf, out_hbm.at[cid, pl.ds(row0, _WB_ROWS)])

            return carry

        lax.fori_loop(0, _WB_ITERS, wb_body, 0)

    return scatter


def _sc_scatter(e, receivers, zeros):
    return _sc_scatter_kernel(receivers.shape[0])(e, receivers, zeros)


def kernel(node_features, edge_features, senders, receivers, params):
    steps = params['steps']
    v, p_tab, q_tab = _node_encode(
        node_features, params['node_enc'],
        steps[0]['edge']['w1'][D:2 * D], steps[0]['edge']['w1'][2 * D:3 * D])
    zeros = jnp.zeros((N, D), jnp.float32)
    nsplit = 4
    h = E // nsplit
    s_c = [senders[k * h:(k + 1) * h] for k in range(nsplit)]
    r_c = [receivers[k * h:(k + 1) * h] for k in range(nsplit)]
    ef_c = [edge_features[k * h:(k + 1) * h] for k in range(nsplit)]
    e_c = [None] * nsplit
    for i, sp in enumerate(steps):
        aggs = []
        for k in range(nsplit):
            g = _sc_gather(p_tab, q_tab, s_c[k], r_c[k])
            if i == 0:
                e_c[k] = _edge_step(None, g, sp['edge'],
                                    enc=params['edge_enc'], raw=ef_c[k])
            else:
                e_c[k] = _edge_step(e_c[k], g, sp['edge'])
            aggs.append(_sc_scatter(e_c[k], r_c[k], zeros))
        nxt = steps[i + 1]['edge'] if i + 1 < len(steps) else None
        v, p_tab, q_tab = _node_step(v, aggs, sp['node'], nxt)
    return _decode(v, params['decoder'])
